# Initial kernel scaffold; baseline (speedup 1.0000x reference)
import jax
import jax.numpy as jnp
from jax.experimental import pallas as pl


def kernel(x, W1, b1, W2, b2, W3, b3, W4, b4, Wfc1, bfc1, Wfc2, bfc2, L0_val, L1_val, L0_row, L0_col, L1_row, L1_col):
    raise NotImplementedError("write your pallas kernel here")


# speedup vs baseline: 2.3196x; 2.3196x over previous
import functools

import jax
import jax.numpy as jnp
from jax import lax
from jax.experimental import pallas as pl
from jax.experimental.pallas import tpu as pltpu
from jax.experimental.pallas import tpu_sc as plsc

V0 = 16384
V1 = 4096
DEG = 8
B = 16
FIN = 3
NOUT = 72

_HIGH = lax.Precision.HIGHEST


_SC_CORES = 2
_SC_SUBCORES = 16


def _make_spmm(V, C, R, fuse):
    NW = _SC_CORES * _SC_SUBCORES
    rows_w = V // NW
    nch = rows_w // R
    E = R * DEG
    NCC = C // 16

    mesh = plsc.VectorSubcoreMesh(core_axis_name="c", subcore_axis_name="s",
                                  num_cores=_SC_CORES, num_subcores=_SC_SUBCORES)
    scratch = [
        pltpu.VMEM((E,), jnp.int32),
        pltpu.VMEM((E,), jnp.float32),
        pltpu.VMEM((E, C), jnp.float32),
        pltpu.VMEM((R, C), jnp.float32),
    ]
    if fuse:
        scratch.append(pltpu.VMEM((R, C), jnp.float32))
    scratch.append(pltpu.SemaphoreType.DMA)

    def body(*refs):
        if fuse:
            (x_hbm, col_hbm, val_hbm, x0_hbm, out_hbm,
             colv, valv, gath, outv, x0v, sem) = refs
        else:
            (x_hbm, col_hbm, val_hbm, out_hbm,
             colv, valv, gath, outv, sem) = refs
        w = lax.axis_index("s") * _SC_CORES + lax.axis_index("c")
        row0w = w * rows_w

        def chunk(i, _):
            r0 = row0w + i * R
            e0 = r0 * DEG
            pltpu.sync_copy(col_hbm.at[pl.ds(e0, E)], colv)
            pltpu.sync_copy(val_hbm.at[pl.ds(e0, E)], valv)
            pltpu.async_copy(x_hbm.at[colv], gath, sem).wait()
            if fuse:
                pltpu.sync_copy(x0_hbm.at[pl.ds(r0, R)], x0v)

            def rowpair(rr, _):
                vv = valv[pl.ds(rr * 16, 16)]
                for half in range(2):
                    r = rr * 2 + half
                    base = r * DEG
                    vals = [vv[half * DEG + d] for d in range(DEG)]
                    for cc in range(NCC):
                        cs = pl.ds(cc * 16, 16)
                        acc = vals[0] * gath[base, cs]
                        for d in range(1, DEG):
                            acc = acc + vals[d] * gath[base + d, cs]
                        if fuse:
                            acc = 2.0 * acc - x0v[r, cs]
                        outv[r, cs] = acc
                return 0

            lax.fori_loop(0, R // 2, rowpair, 0)
            pltpu.sync_copy(outv, out_hbm.at[pl.ds(r0, R)])
            return 0

        lax.fori_loop(0, nch, chunk, 0)

    kparams = pltpu.CompilerParams(use_tc_tiling_on_sc=False)
    if fuse:
        def run(X, col, val, Xprev):
            return pl.kernel(
                body,
                out_type=jax.ShapeDtypeStruct((V, C), jnp.float32),
                mesh=mesh,
                scratch_types=scratch,
                compiler_params=kparams,
            )(X, col, val, Xprev)
    else:
        def run(X, col, val):
            return pl.kernel(
                body,
                out_type=jax.ShapeDtypeStruct((V, C), jnp.float32),
                mesh=mesh,
                scratch_types=scratch,
                compiler_params=kparams,
            )(X, col, val)
    return run


def _linear3(x0, x1, x2, w0, w1, w2, b, F, O, pool, blk=4096):
    M = x0.shape[0]
    grid = M // blk
    oblk = blk // 4 if pool else blk

    def body(x0_ref, x1_ref, x2_ref, w0_ref, w1_ref, w2_ref, b_ref, o_ref):
        if F < 8:
            y = b_ref[...]
            for f in range(F):
                y = (y + x0_ref[:, f:f + 1] * w0_ref[f:f + 1, :]
                     + x1_ref[:, f:f + 1] * w1_ref[f:f + 1, :]
                     + x2_ref[:, f:f + 1] * w2_ref[f:f + 1, :])
        else:
            y = lax.dot_general(x0_ref[...], w0_ref[...], (((1,), (0,)), ((), ())),
                                precision=_HIGH, preferred_element_type=jnp.float32)
            y = y + lax.dot_general(x1_ref[...], w1_ref[...], (((1,), (0,)), ((), ())),
                                    precision=_HIGH, preferred_element_type=jnp.float32)
            y = y + lax.dot_general(x2_ref[...], w2_ref[...], (((1,), (0,)), ((), ())),
                                    precision=_HIGH, preferred_element_type=jnp.float32)
            y = y + b_ref[...]
        if pool:
            y = y.reshape(blk // (4 * B), 4, B, O).max(axis=1).reshape(oblk, O)
        o_ref[...] = y

    wspec = pl.BlockSpec((F, O), lambda i: (0, 0))
    return pl.pallas_call(
        body,
        grid=(grid,),
        in_specs=[
            pl.BlockSpec((blk, F), lambda i: (i, 0)),
            pl.BlockSpec((blk, F), lambda i: (i, 0)),
            pl.BlockSpec((blk, F), lambda i: (i, 0)),
            wspec, wspec, wspec,
            pl.BlockSpec((1, O), lambda i: (0, 0)),
        ],
        out_specs=pl.BlockSpec((oblk, O), lambda i: (i, 0)),
        out_shape=jax.ShapeDtypeStruct((M // 4 if pool else M, O), jnp.float32),
    )(x0, x1, x2, w0, w1, w2, b)


def _fc(Z2, Wfc1, bfc1, Wfc2, bfc2, ch=32):
    nv = Z2.shape[0] // B
    grid = nv // ch

    def body(z_ref, w1_ref, b1_ref, w2_ref, b2_ref, o_ref, acc_ref):
        i = pl.program_id(0)

        @pl.when(i == 0)
        def _():
            acc_ref[...] = jnp.zeros_like(acc_ref)

        acc = acc_ref[...]
        for j in range(ch):
            zz = z_ref[j * B:(j + 1) * B, :]
            wc = w1_ref[:, j * 64:(j + 1) * 64]
            acc = acc + lax.dot_general(zz, wc, (((1,), (1,)), ((), ())),
                                        precision=_HIGH,
                                        preferred_element_type=jnp.float32)
        acc_ref[...] = acc

        @pl.when(i == grid - 1)
        def _():
            h1 = acc_ref[...] + b1_ref[...]
            out = lax.dot_general(h1, w2_ref[...], (((1,), (1,)), ((), ())),
                                  precision=_HIGH,
                                  preferred_element_type=jnp.float32)
            o_ref[...] = out + b2_ref[...]

    return pl.pallas_call(
        body,
        grid=(grid,),
        in_specs=[
            pl.BlockSpec((ch * B, 64), lambda i: (i, 0)),
            pl.BlockSpec((512, ch * 64), lambda i: (0, i)),
            pl.BlockSpec((1, 512), lambda i: (0, 0)),
            pl.BlockSpec((NOUT, 512), lambda i: (0, 0)),
            pl.BlockSpec((1, NOUT), lambda i: (0, 0)),
        ],
        out_specs=pl.BlockSpec((B, NOUT), lambda i: (0, 0)),
        out_shape=jax.ShapeDtypeStruct((B, NOUT), jnp.float32),
        scratch_shapes=[pltpu.VMEM((B, 512), jnp.float32)],
    )(Z2, Wfc1, bfc1, Wfc2, bfc2)


def _wk(W, F, O):
    return [W[:, k::3].T for k in range(3)]


def kernel(x, W1, b1, W2, b2, W3, b3, W4, b4, Wfc1, bfc1, Wfc2, bfc2,
           L0_val, L1_val, L0_row, L0_col, L1_row, L1_col):
    del L0_row, L1_row

    spmm1 = _make_spmm(V0, B * FIN, 16, False)
    spmm1f = _make_spmm(V0, B * FIN, 16, True)
    spmm2 = _make_spmm(V0, B * 32, 16, False)
    spmm2f = _make_spmm(V0, B * 32, 16, True)
    spmm3 = _make_spmm(V1, B * 32, 16, False)
    spmm3f = _make_spmm(V1, B * 32, 16, True)
    spmm4 = _make_spmm(V1, B * 64, 8, False)
    spmm4f = _make_spmm(V1, B * 64, 8, True)

    X0 = jnp.transpose(x, (1, 0, 2)).reshape(V0, B * FIN)
    X1 = spmm1(X0, L0_col, L0_val)
    X2 = spmm1f(X1, L0_col, L0_val, X0)
    k0, k1, k2 = _wk(W1, FIN, 32)
    H = _linear3(X0.reshape(-1, FIN), X1.reshape(-1, FIN), X2.reshape(-1, FIN),
                 k0, k1, k2, b1.reshape(1, -1), FIN, 32, False)
    H = H.reshape(V0, B * 32)

    X1 = spmm2(H, L0_col, L0_val)
    X2 = spmm2f(X1, L0_col, L0_val, H)
    k0, k1, k2 = _wk(W2, 32, 32)
    H = _linear3(H.reshape(-1, 32), X1.reshape(-1, 32), X2.reshape(-1, 32),
                 k0, k1, k2, b2.reshape(1, -1), 32, 32, True)
    H = H.reshape(V1, B * 32)

    X1 = spmm3(H, L1_col, L1_val)
    X2 = spmm3f(X1, L1_col, L1_val, H)
    k0, k1, k2 = _wk(W3, 32, 64)
    H = _linear3(H.reshape(-1, 32), X1.reshape(-1, 32), X2.reshape(-1, 32),
                 k0, k1, k2, b3.reshape(1, -1), 32, 64, False)
    H = H.reshape(V1, B * 64)

    X1 = spmm4(H, L1_col, L1_val)
    X2 = spmm4f(X1, L1_col, L1_val, H)
    k0, k1, k2 = _wk(W4, 64, 64)
    Z = _linear3(H.reshape(-1, 64), X1.reshape(-1, 64), X2.reshape(-1, 64),
                 k0, k1, k2, b4.reshape(1, -1), 64, 64, True)

    Z2 = Z.reshape(B * 1024, 64)
    return _fc(Z2, Wfc1, bfc1.reshape(1, -1), Wfc2, bfc2.reshape(1, -1))


# speedup vs baseline: 3.0084x; 1.2969x over previous
import functools

import jax
import jax.numpy as jnp
from jax import lax
from jax.experimental import pallas as pl
from jax.experimental.pallas import tpu as pltpu
from jax.experimental.pallas import tpu_sc as plsc

V0 = 16384
V1 = 4096
DEG = 8
B = 16
FIN = 3
NOUT = 72

_HIGH = lax.Precision.HIGHEST


_SC_CORES = 2
_SC_SUBCORES = 16


def _make_spmm(V, C, R, fuse):
    NW = _SC_CORES * _SC_SUBCORES
    rows_w = V // NW
    nch = rows_w // R
    assert nch % 2 == 0
    E = R * DEG
    NCC = C // 16
    EW = rows_w * DEG

    mesh = plsc.VectorSubcoreMesh(core_axis_name="c", subcore_axis_name="s",
                                  num_cores=_SC_CORES, num_subcores=_SC_SUBCORES)
    scratch = [
        pltpu.VMEM((EW,), jnp.int32),
        pltpu.VMEM((EW,), jnp.float32),
        pltpu.VMEM((2, E, C), jnp.float32),
        pltpu.VMEM((2, R, C), jnp.float32),
    ]
    if fuse:
        scratch.append(pltpu.VMEM((2, R, C), jnp.float32))
    scratch.extend([pltpu.SemaphoreType.DMA] * (6 if fuse else 4))

    def body(*refs):
        if fuse:
            (x_hbm, col_hbm, val_hbm, x0_hbm, out_hbm,
             colv, valv, gath, outv, x0v, g0, g1, o0, o1, xs0, xs1) = refs
            xsem = (xs0, xs1)
        else:
            (x_hbm, col_hbm, val_hbm, out_hbm,
             colv, valv, gath, outv, g0, g1, o0, o1) = refs
        gsem = (g0, g1)
        osem = (o0, o1)
        w = lax.axis_index("s") * _SC_CORES + lax.axis_index("c")
        row0w = w * rows_w

        pltpu.sync_copy(col_hbm.at[pl.ds(row0w * DEG, EW)], colv)
        pltpu.sync_copy(val_hbm.at[pl.ds(row0w * DEG, EW)], valv)

        def g_desc(ci, slot):
            return pltpu.make_async_copy(
                x_hbm.at[colv.at[pl.ds(ci * E, E)]], gath.at[slot], gsem[slot])

        def x_desc(ci, slot):
            return pltpu.make_async_copy(
                x0_hbm.at[pl.ds(row0w + ci * R, R)], x0v.at[slot], xsem[slot])

        def o_desc(ci, slot):
            return pltpu.make_async_copy(
                outv.at[slot], out_hbm.at[pl.ds(row0w + ci * R, R)], osem[slot])

        g_desc(0, 0).start()
        if fuse:
            x_desc(0, 0).start()

        def step(ci, slot):
            @pl.when(ci + 1 < nch)
            def _():
                g_desc(ci + 1, 1 - slot).start()
                if fuse:
                    x_desc(ci + 1, 1 - slot).start()

            @pl.when(ci >= 2)
            def _():
                o_desc(ci - 2, slot).wait()

            g_desc(ci, slot).wait()
            if fuse:
                x_desc(ci, slot).wait()
            gb = gath.at[slot]
            ob = outv.at[slot]
            if fuse:
                xb = x0v.at[slot]

            def rowpair(rr, _):
                vv = valv[pl.ds(ci * E + rr * 16, 16)]
                for half in range(2):
                    r = rr * 2 + half
                    base = r * DEG
                    vals = [vv[half * DEG + d] for d in range(DEG)]
                    for cc in range(NCC):
                        cs = pl.ds(cc * 16, 16)
                        acc = vals[0] * gb[base, cs]
                        for d in range(1, DEG):
                            acc = acc + vals[d] * gb[base + d, cs]
                        if fuse:
                            acc = 2.0 * acc - xb[r, cs]
                        ob[r, cs] = acc
                return 0

            lax.fori_loop(0, R // 2, rowpair, 0)
            o_desc(ci, slot).start()

        def loop_j(j, _):
            step(j * 2, 0)
            step(j * 2 + 1, 1)
            return 0

        lax.fori_loop(0, nch // 2, loop_j, 0)
        o_desc(nch - 2, 0).wait()
        o_desc(nch - 1, 1).wait()

    kparams = pltpu.CompilerParams(use_tc_tiling_on_sc=False)
    if fuse:
        def run(X, col, val, Xprev):
            return pl.kernel(
                body,
                out_type=jax.ShapeDtypeStruct((V, C), jnp.float32),
                mesh=mesh,
                scratch_types=scratch,
                compiler_params=kparams,
            )(X, col, val, Xprev)
    else:
        def run(X, col, val):
            return pl.kernel(
                body,
                out_type=jax.ShapeDtypeStruct((V, C), jnp.float32),
                mesh=mesh,
                scratch_types=scratch,
                compiler_params=kparams,
            )(X, col, val)
    return run


def _linear1(x0, x1, x2, w0, w1, w2, b, blk=2048):
    V, C = x0.shape
    O = w0.shape[1]

    def body(x0_ref, x1_ref, x2_ref, w0_ref, w1_ref, w2_ref, b_ref, o_ref):
        y = lax.dot_general(x0_ref[...], w0_ref[...], (((1,), (0,)), ((), ())),
                            precision=_HIGH, preferred_element_type=jnp.float32)
        y = y + lax.dot_general(x1_ref[...], w1_ref[...], (((1,), (0,)), ((), ())),
                                precision=_HIGH, preferred_element_type=jnp.float32)
        y = y + lax.dot_general(x2_ref[...], w2_ref[...], (((1,), (0,)), ((), ())),
                                precision=_HIGH, preferred_element_type=jnp.float32)
        o_ref[...] = y + b_ref[...]

    wspec = pl.BlockSpec((C, O), lambda i: (0, 0))
    return pl.pallas_call(
        body,
        grid=(V // blk,),
        in_specs=[
            pl.BlockSpec((blk, C), lambda i: (i, 0)),
            pl.BlockSpec((blk, C), lambda i: (i, 0)),
            pl.BlockSpec((blk, C), lambda i: (i, 0)),
            wspec, wspec, wspec,
            pl.BlockSpec((1, O), lambda i: (0, 0)),
        ],
        out_specs=pl.BlockSpec((blk, O), lambda i: (i, 0)),
        out_shape=jax.ShapeDtypeStruct((V, O), jnp.float32),
    )(x0, x1, x2, w0, w1, w2, b)


def _linear3(x0, x1, x2, w0, w1, w2, b, F, O, pool, blk=4096):
    M = x0.shape[0]
    grid = M // blk
    oblk = blk // 4 if pool else blk

    def body(x0_ref, x1_ref, x2_ref, w0_ref, w1_ref, w2_ref, b_ref, o_ref):
        if F < 8:
            y = b_ref[...]
            for f in range(F):
                y = (y + x0_ref[:, f:f + 1] * w0_ref[f:f + 1, :]
                     + x1_ref[:, f:f + 1] * w1_ref[f:f + 1, :]
                     + x2_ref[:, f:f + 1] * w2_ref[f:f + 1, :])
        else:
            y = lax.dot_general(x0_ref[...], w0_ref[...], (((1,), (0,)), ((), ())),
                                precision=_HIGH, preferred_element_type=jnp.float32)
            y = y + lax.dot_general(x1_ref[...], w1_ref[...], (((1,), (0,)), ((), ())),
                                    precision=_HIGH, preferred_element_type=jnp.float32)
            y = y + lax.dot_general(x2_ref[...], w2_ref[...], (((1,), (0,)), ((), ())),
                                    precision=_HIGH, preferred_element_type=jnp.float32)
            y = y + b_ref[...]
        if pool:
            y = y.reshape(blk // (4 * B), 4, B, O).max(axis=1).reshape(oblk, O)
        o_ref[...] = y

    wspec = pl.BlockSpec((F, O), lambda i: (0, 0))
    return pl.pallas_call(
        body,
        grid=(grid,),
        in_specs=[
            pl.BlockSpec((blk, F), lambda i: (i, 0)),
            pl.BlockSpec((blk, F), lambda i: (i, 0)),
            pl.BlockSpec((blk, F), lambda i: (i, 0)),
            wspec, wspec, wspec,
            pl.BlockSpec((1, O), lambda i: (0, 0)),
        ],
        out_specs=pl.BlockSpec((oblk, O), lambda i: (i, 0)),
        out_shape=jax.ShapeDtypeStruct((M // 4 if pool else M, O), jnp.float32),
    )(x0, x1, x2, w0, w1, w2, b)


def _fc(Z2, Wfc1, bfc1, Wfc2, bfc2, ch=32):
    nv = Z2.shape[0] // B
    grid = nv // ch

    def body(z_ref, w1_ref, b1_ref, w2_ref, b2_ref, o_ref, acc_ref):
        i = pl.program_id(0)

        @pl.when(i == 0)
        def _():
            acc_ref[...] = jnp.zeros_like(acc_ref)

        acc = acc_ref[...]
        for j in range(ch):
            zz = z_ref[j * B:(j + 1) * B, :]
            wc = w1_ref[:, j * 64:(j + 1) * 64]
            acc = acc + lax.dot_general(zz, wc, (((1,), (1,)), ((), ())),
                                        precision=_HIGH,
                                        preferred_element_type=jnp.float32)
        acc_ref[...] = acc

        @pl.when(i == grid - 1)
        def _():
            h1 = acc_ref[...] + b1_ref[...]
            out = lax.dot_general(h1, w2_ref[...], (((1,), (1,)), ((), ())),
                                  precision=_HIGH,
                                  preferred_element_type=jnp.float32)
            o_ref[...] = out + b2_ref[...]

    return pl.pallas_call(
        body,
        grid=(grid,),
        in_specs=[
            pl.BlockSpec((ch * B, 64), lambda i: (i, 0)),
            pl.BlockSpec((512, ch * 64), lambda i: (0, i)),
            pl.BlockSpec((1, 512), lambda i: (0, 0)),
            pl.BlockSpec((NOUT, 512), lambda i: (0, 0)),
            pl.BlockSpec((1, NOUT), lambda i: (0, 0)),
        ],
        out_specs=pl.BlockSpec((B, NOUT), lambda i: (0, 0)),
        out_shape=jax.ShapeDtypeStruct((B, NOUT), jnp.float32),
        scratch_shapes=[pltpu.VMEM((B, 512), jnp.float32)],
    )(Z2, Wfc1, bfc1, Wfc2, bfc2)


def _wk(W, F, O):
    return [W[:, k::3].T for k in range(3)]


def kernel(x, W1, b1, W2, b2, W3, b3, W4, b4, Wfc1, bfc1, Wfc2, bfc2,
           L0_val, L1_val, L0_row, L0_col, L1_row, L1_col):
    del L0_row, L1_row

    spmm1 = _make_spmm(V0, B * FIN, 16, False)
    spmm1f = _make_spmm(V0, B * FIN, 16, True)
    spmm2 = _make_spmm(V0, B * 32, 8, False)
    spmm2f = _make_spmm(V0, B * 32, 8, True)
    spmm3 = _make_spmm(V1, B * 32, 8, False)
    spmm3f = _make_spmm(V1, B * 32, 8, True)
    spmm4 = _make_spmm(V1, B * 64, 4, False)
    spmm4f = _make_spmm(V1, B * 64, 4, True)

    X0 = jnp.transpose(x, (1, 0, 2)).reshape(V0, B * FIN)
    X1 = spmm1(X0, L0_col, L0_val)
    X2 = spmm1f(X1, L0_col, L0_val, X0)
    k0, k1, k2 = _wk(W1, FIN, 32)
    eyeB = jnp.eye(B, dtype=jnp.float32)
    H = _linear1(X0, X1, X2,
                 jnp.kron(eyeB, k0), jnp.kron(eyeB, k1), jnp.kron(eyeB, k2),
                 jnp.tile(b1, B).reshape(1, -1))
    H = H.reshape(V0, B * 32)

    X1 = spmm2(H, L0_col, L0_val)
    X2 = spmm2f(X1, L0_col, L0_val, H)
    k0, k1, k2 = _wk(W2, 32, 32)
    H = _linear3(H.reshape(-1, 32), X1.reshape(-1, 32), X2.reshape(-1, 32),
                 k0, k1, k2, b2.reshape(1, -1), 32, 32, True)
    H = H.reshape(V1, B * 32)

    X1 = spmm3(H, L1_col, L1_val)
    X2 = spmm3f(X1, L1_col, L1_val, H)
    k0, k1, k2 = _wk(W3, 32, 64)
    H = _linear3(H.reshape(-1, 32), X1.reshape(-1, 32), X2.reshape(-1, 32),
                 k0, k1, k2, b3.reshape(1, -1), 32, 64, False)
    H = H.reshape(V1, B * 64)

    X1 = spmm4(H, L1_col, L1_val)
    X2 = spmm4f(X1, L1_col, L1_val, H)
    k0, k1, k2 = _wk(W4, 64, 64)
    Z = _linear3(H.reshape(-1, 64), X1.reshape(-1, 64), X2.reshape(-1, 64),
                 k0, k1, k2, b4.reshape(1, -1), 64, 64, True)

    Z2 = Z.reshape(B * 1024, 64)
    return _fc(Z2, Wfc1, bfc1.reshape(1, -1), Wfc2, bfc2.reshape(1, -1))


# speedup vs baseline: 3.4505x; 1.1470x over previous
import functools

import jax
import jax.numpy as jnp
from jax import lax
from jax.experimental import pallas as pl
from jax.experimental.pallas import tpu as pltpu
from jax.experimental.pallas import tpu_sc as plsc

V0 = 16384
V1 = 4096
DEG = 8
B = 16
FIN = 3
NOUT = 72

_HIGH = lax.Precision.HIGHEST


_SC_CORES = 2
_SC_SUBCORES = 16


def _make_spmm(V, C, CC, R, fuse):
    NW = _SC_CORES * _SC_SUBCORES
    NS = _SC_SUBCORES
    rows_w = V // NW
    nch = rows_w // R
    assert nch % 2 == 0
    E = R * DEG
    P = C // CC
    NCC = CC // 16
    EW = rows_w * DEG
    VT = V // NS

    mesh = plsc.VectorSubcoreMesh(core_axis_name="c", subcore_axis_name="s",
                                  num_cores=_SC_CORES, num_subcores=_SC_SUBCORES)
    scratch = [
        pltpu.VMEM((EW,), jnp.int32),
        pltpu.VMEM((EW,), jnp.float32),
        pltpu.VMEM_SHARED((V, CC), jnp.float32),
        pltpu.VMEM((2, E, CC), jnp.float32),
        pltpu.VMEM((2, R, CC), jnp.float32),
    ]
    if fuse:
        scratch.append(pltpu.VMEM((2, R, CC), jnp.float32))
    scratch.extend([pltpu.SemaphoreType.DMA] * (7 if fuse else 5))

    def body(*refs):
        if fuse:
            (x_hbm, col_hbm, val_hbm, x0_hbm, out_hbm,
             colv, valv, tab, gath, outv, x0v,
             g0, g1, o0, o1, xs0, xs1, ssem) = refs
            xsem = (xs0, xs1)
        else:
            (x_hbm, col_hbm, val_hbm, out_hbm,
             colv, valv, tab, gath, outv,
             g0, g1, o0, o1, ssem) = refs
        gsem = (g0, g1)
        osem = (o0, o1)
        sid = lax.axis_index("s")
        w = sid * _SC_CORES + lax.axis_index("c")
        row0w = w * rows_w
        row0t = sid * VT

        pltpu.sync_copy(col_hbm.at[pl.ds(row0w * DEG, EW)], colv)
        pltpu.sync_copy(val_hbm.at[pl.ds(row0w * DEG, EW)], valv)

        def run_pass(p, _):
            pc0 = p * CC

            pltpu.async_copy(
                x_hbm.at[pl.ds(row0t, VT), pl.ds(pc0, CC)],
                tab.at[pl.ds(row0t, VT)], ssem).wait()
            plsc.subcore_barrier()

            def x_desc(ci, slot):
                return pltpu.make_async_copy(
                    x0_hbm.at[pl.ds(row0w + ci * R, R), pl.ds(pc0, CC)],
                    x0v.at[slot], xsem[slot])

            def o_desc(ci, slot):
                return pltpu.make_async_copy(
                    outv.at[slot],
                    out_hbm.at[pl.ds(row0w + ci * R, R), pl.ds(pc0, CC)],
                    osem[slot])

            def g_desc(ci, slot):
                return pltpu.make_async_copy(
                    tab.at[colv.at[pl.ds(ci * E, E)]], gath.at[slot],
                    gsem[slot])

            g_desc(0, 0).start()
            if fuse:
                x_desc(0, 0).start()

            def step(ci, slot):
                @pl.when(ci + 1 < nch)
                def _():
                    g_desc(ci + 1, 1 - slot).start()
                    if fuse:
                        x_desc(ci + 1, 1 - slot).start()

                @pl.when(ci >= 2)
                def _():
                    o_desc(ci - 2, slot).wait()

                g_desc(ci, slot).wait()
                if fuse:
                    x_desc(ci, slot).wait()
                gb = gath.at[slot]
                ob = outv.at[slot]
                if fuse:
                    xb = x0v.at[slot]

                def rowpair(rr, _):
                    vv = valv[pl.ds(ci * E + rr * 16, 16)]
                    for half in range(2):
                        r = rr * 2 + half
                        base = r * DEG
                        vals = [vv[half * DEG + d] for d in range(DEG)]
                        for cc in range(NCC):
                            cs = pl.ds(cc * 16, 16)
                            acc = vals[0] * gb[base, cs]
                            for d in range(1, DEG):
                                acc = acc + vals[d] * gb[base + d, cs]
                            if fuse:
                                acc = 2.0 * acc - xb[r, cs]
                            ob[r, cs] = acc
                    return 0

                lax.fori_loop(0, R // 2, rowpair, 0)
                o_desc(ci, slot).start()

            def loop_j(j, _):
                step(j * 2, 0)
                step(j * 2 + 1, 1)
                return 0

            lax.fori_loop(0, nch // 2, loop_j, 0)
            o_desc(nch - 2, 0).wait()
            o_desc(nch - 1, 1).wait()
            plsc.subcore_barrier()
            return 0

        lax.fori_loop(0, P, run_pass, 0)

    kparams = pltpu.CompilerParams(use_tc_tiling_on_sc=False)
    if fuse:
        def run(X, col, val, Xprev):
            return pl.kernel(
                body,
                out_type=jax.ShapeDtypeStruct((V, C), jnp.float32),
                mesh=mesh,
                scratch_types=scratch,
                compiler_params=kparams,
            )(X, col, val, Xprev)
    else:
        def run(X, col, val):
            return pl.kernel(
                body,
                out_type=jax.ShapeDtypeStruct((V, C), jnp.float32),
                mesh=mesh,
                scratch_types=scratch,
                compiler_params=kparams,
            )(X, col, val)
    return run


def _linear1(x0, x1, x2, w0, w1, w2, b, blk=2048):
    V, C = x0.shape
    O = w0.shape[1]

    def body(x0_ref, x1_ref, x2_ref, w0_ref, w1_ref, w2_ref, b_ref, o_ref):
        y = lax.dot_general(x0_ref[...], w0_ref[...], (((1,), (0,)), ((), ())),
                            precision=_HIGH, preferred_element_type=jnp.float32)
        y = y + lax.dot_general(x1_ref[...], w1_ref[...], (((1,), (0,)), ((), ())),
                                precision=_HIGH, preferred_element_type=jnp.float32)
        y = y + lax.dot_general(x2_ref[...], w2_ref[...], (((1,), (0,)), ((), ())),
                                precision=_HIGH, preferred_element_type=jnp.float32)
        o_ref[...] = y + b_ref[...]

    wspec = pl.BlockSpec((C, O), lambda i: (0, 0))
    return pl.pallas_call(
        body,
        grid=(V // blk,),
        in_specs=[
            pl.BlockSpec((blk, C), lambda i: (i, 0)),
            pl.BlockSpec((blk, C), lambda i: (i, 0)),
            pl.BlockSpec((blk, C), lambda i: (i, 0)),
            wspec, wspec, wspec,
            pl.BlockSpec((1, O), lambda i: (0, 0)),
        ],
        out_specs=pl.BlockSpec((blk, O), lambda i: (i, 0)),
        out_shape=jax.ShapeDtypeStruct((V, O), jnp.float32),
    )(x0, x1, x2, w0, w1, w2, b)


def _linear3(x0, x1, x2, w0, w1, w2, b, F, O, pool, blk=4096):
    M = x0.shape[0]
    grid = M // blk
    oblk = blk // 4 if pool else blk

    def body(x0_ref, x1_ref, x2_ref, w0_ref, w1_ref, w2_ref, b_ref, o_ref):
        if F < 8:
            y = b_ref[...]
            for f in range(F):
                y = (y + x0_ref[:, f:f + 1] * w0_ref[f:f + 1, :]
                     + x1_ref[:, f:f + 1] * w1_ref[f:f + 1, :]
                     + x2_ref[:, f:f + 1] * w2_ref[f:f + 1, :])
        else:
            y = lax.dot_general(x0_ref[...], w0_ref[...], (((1,), (0,)), ((), ())),
                                precision=_HIGH, preferred_element_type=jnp.float32)
            y = y + lax.dot_general(x1_ref[...], w1_ref[...], (((1,), (0,)), ((), ())),
                                    precision=_HIGH, preferred_element_type=jnp.float32)
            y = y + lax.dot_general(x2_ref[...], w2_ref[...], (((1,), (0,)), ((), ())),
                                    precision=_HIGH, preferred_element_type=jnp.float32)
            y = y + b_ref[...]
        if pool:
            y = y.reshape(blk // (4 * B), 4, B, O).max(axis=1).reshape(oblk, O)
        o_ref[...] = y

    wspec = pl.BlockSpec((F, O), lambda i: (0, 0))
    return pl.pallas_call(
        body,
        grid=(grid,),
        in_specs=[
            pl.BlockSpec((blk, F), lambda i: (i, 0)),
            pl.BlockSpec((blk, F), lambda i: (i, 0)),
            pl.BlockSpec((blk, F), lambda i: (i, 0)),
            wspec, wspec, wspec,
            pl.BlockSpec((1, O), lambda i: (0, 0)),
        ],
        out_specs=pl.BlockSpec((oblk, O), lambda i: (i, 0)),
        out_shape=jax.ShapeDtypeStruct((M // 4 if pool else M, O), jnp.float32),
    )(x0, x1, x2, w0, w1, w2, b)


def _fc(Z2, Wfc1, bfc1, Wfc2, bfc2, ch=32):
    nv = Z2.shape[0] // B
    grid = nv // ch

    def body(z_ref, w1_ref, b1_ref, w2_ref, b2_ref, o_ref, acc_ref):
        i = pl.program_id(0)

        @pl.when(i == 0)
        def _():
            acc_ref[...] = jnp.zeros_like(acc_ref)

        acc = acc_ref[...]
        for j in range(ch):
            zz = z_ref[j * B:(j + 1) * B, :]
            wc = w1_ref[:, j * 64:(j + 1) * 64]
            acc = acc + lax.dot_general(zz, wc, (((1,), (1,)), ((), ())),
                                        precision=_HIGH,
                                        preferred_element_type=jnp.float32)
        acc_ref[...] = acc

        @pl.when(i == grid - 1)
        def _():
            h1 = acc_ref[...] + b1_ref[...]
            out = lax.dot_general(h1, w2_ref[...], (((1,), (1,)), ((), ())),
                                  precision=_HIGH,
                                  preferred_element_type=jnp.float32)
            o_ref[...] = out + b2_ref[...]

    return pl.pallas_call(
        body,
        grid=(grid,),
        in_specs=[
            pl.BlockSpec((ch * B, 64), lambda i: (i, 0)),
            pl.BlockSpec((512, ch * 64), lambda i: (0, i)),
            pl.BlockSpec((1, 512), lambda i: (0, 0)),
            pl.BlockSpec((NOUT, 512), lambda i: (0, 0)),
            pl.BlockSpec((1, NOUT), lambda i: (0, 0)),
        ],
        out_specs=pl.BlockSpec((B, NOUT), lambda i: (0, 0)),
        out_shape=jax.ShapeDtypeStruct((B, NOUT), jnp.float32),
        scratch_shapes=[pltpu.VMEM((B, 512), jnp.float32)],
    )(Z2, Wfc1, bfc1, Wfc2, bfc2)


def _wk(W, F, O):
    return [W[:, k::3].T for k in range(3)]


def kernel(x, W1, b1, W2, b2, W3, b3, W4, b4, Wfc1, bfc1, Wfc2, bfc2,
           L0_val, L1_val, L0_row, L0_col, L1_row, L1_col):
    del L0_row, L1_row

    spmm1 = _make_spmm(V0, B * FIN, 48, 16, False)
    spmm1f = _make_spmm(V0, B * FIN, 48, 16, True)
    spmm2 = _make_spmm(V0, B * 32, 32, 16, False)
    spmm2f = _make_spmm(V0, B * 32, 32, 16, True)
    spmm3 = _make_spmm(V1, B * 32, 128, 16, False)
    spmm3f = _make_spmm(V1, B * 32, 128, 16, True)
    spmm4 = _make_spmm(V1, B * 64, 128, 16, False)
    spmm4f = _make_spmm(V1, B * 64, 128, 16, True)

    X0 = jnp.transpose(x, (1, 0, 2)).reshape(V0, B * FIN)
    X1 = spmm1(X0, L0_col, L0_val)
    X2 = spmm1f(X1, L0_col, L0_val, X0)
    k0, k1, k2 = _wk(W1, FIN, 32)
    eyeB = jnp.eye(B, dtype=jnp.float32)
    H = _linear1(X0, X1, X2,
                 jnp.kron(eyeB, k0), jnp.kron(eyeB, k1), jnp.kron(eyeB, k2),
                 jnp.tile(b1, B).reshape(1, -1))
    H = H.reshape(V0, B * 32)

    X1 = spmm2(H, L0_col, L0_val)
    X2 = spmm2f(X1, L0_col, L0_val, H)
    k0, k1, k2 = _wk(W2, 32, 32)
    H = _linear3(H.reshape(-1, 32), X1.reshape(-1, 32), X2.reshape(-1, 32),
                 k0, k1, k2, b2.reshape(1, -1), 32, 32, True)
    H = H.reshape(V1, B * 32)

    X1 = spmm3(H, L1_col, L1_val)
    X2 = spmm3f(X1, L1_col, L1_val, H)
    k0, k1, k2 = _wk(W3, 32, 64)
    H = _linear3(H.reshape(-1, 32), X1.reshape(-1, 32), X2.reshape(-1, 32),
                 k0, k1, k2, b3.reshape(1, -1), 32, 64, False)
    H = H.reshape(V1, B * 64)

    X1 = spmm4(H, L1_col, L1_val)
    X2 = spmm4f(X1, L1_col, L1_val, H)
    k0, k1, k2 = _wk(W4, 64, 64)
    Z = _linear3(H.reshape(-1, 64), X1.reshape(-1, 64), X2.reshape(-1, 64),
                 k0, k1, k2, b4.reshape(1, -1), 64, 64, True)

    Z2 = Z.reshape(B * 1024, 64)
    return _fc(Z2, Wfc1, bfc1.reshape(1, -1), Wfc2, bfc2.reshape(1, -1))


# speedup vs baseline: 3.6838x; 1.0676x over previous
import functools

import jax
import jax.numpy as jnp
from jax import lax
from jax.experimental import pallas as pl
from jax.experimental.pallas import tpu as pltpu
from jax.experimental.pallas import tpu_sc as plsc

V0 = 16384
V1 = 4096
DEG = 8
B = 16
FIN = 3
NOUT = 72

_HIGH = lax.Precision.HIGHEST


_SC_CORES = 2
_SC_SUBCORES = 16


def _make_spmm(V, C, CC, R, fuse):
    NS = _SC_SUBCORES
    P = C // CC
    split = P >= 2
    if split:
        assert P % 2 == 0
        rows_w = V // NS
    else:
        rows_w = V // (2 * NS)
    nch = rows_w // R
    assert nch % 2 == 0
    E = R * DEG
    NSUB = (E + 127) // 128
    assert E % NSUB == 0
    ES = E // NSUB
    RS = ES // DEG
    NCC = CC // 16
    EW = rows_w * DEG
    VT = V // NS

    mesh = plsc.VectorSubcoreMesh(core_axis_name="c", subcore_axis_name="s",
                                  num_cores=_SC_CORES, num_subcores=_SC_SUBCORES)
    scratch = [
        pltpu.VMEM((EW,), jnp.int32),
        pltpu.VMEM((EW,), jnp.float32),
        pltpu.VMEM_SHARED((V, CC), jnp.float32),
        pltpu.VMEM((2, E, CC), jnp.float32),
        pltpu.VMEM((2, R, CC), jnp.float32),
    ]
    if fuse:
        scratch.append(pltpu.VMEM((2, R, CC), jnp.float32))
    scratch.extend([pltpu.SemaphoreType.DMA] * (7 if fuse else 5))

    def body(*refs):
        if fuse:
            (x_hbm, col_hbm, val_hbm, x0_hbm, out_hbm,
             colv, valv, tab, gath, outv, x0v,
             g0, g1, o0, o1, xs0, xs1, ssem) = refs
            xsem = (xs0, xs1)
        else:
            (x_hbm, col_hbm, val_hbm, out_hbm,
             colv, valv, tab, gath, outv,
             g0, g1, o0, o1, ssem) = refs
        gsem = (g0, g1)
        osem = (o0, o1)
        sid = lax.axis_index("s")
        core = lax.axis_index("c")
        if split:
            w = sid
            npass = P // 2
        else:
            w = sid * _SC_CORES + core
            npass = P
        row0w = w * rows_w
        row0t = sid * VT

        pltpu.sync_copy(col_hbm.at[pl.ds(row0w * DEG, EW)], colv)
        pltpu.sync_copy(val_hbm.at[pl.ds(row0w * DEG, EW)], valv)

        def run_pass(j, _):
            p = j * 2 + core if split else j
            pc0 = p * CC

            pltpu.async_copy(
                x_hbm.at[pl.ds(row0t, VT), pl.ds(pc0, CC)],
                tab.at[pl.ds(row0t, VT)], ssem).wait()
            plsc.subcore_barrier()

            def x_desc(ci, slot):
                return pltpu.make_async_copy(
                    x0_hbm.at[pl.ds(row0w + ci * R, R), pl.ds(pc0, CC)],
                    x0v.at[slot], xsem[slot])

            def o_desc(ci, slot):
                return pltpu.make_async_copy(
                    outv.at[slot],
                    out_hbm.at[pl.ds(row0w + ci * R, R), pl.ds(pc0, CC)],
                    osem[slot])

            def g_descs(ci, slot):
                return [
                    pltpu.make_async_copy(
                        tab.at[colv.at[pl.ds(ci * E + s * ES, ES)]],
                        gath.at[slot].at[pl.ds(s * RS * DEG, ES)],
                        gsem[slot])
                    for s in range(NSUB)
                ]

            for d in g_descs(0, 0):
                d.start()
            if fuse:
                x_desc(0, 0).start()

            def step(ci, slot):
                @pl.when(ci + 1 < nch)
                def _():
                    for d in g_descs(ci + 1, 1 - slot):
                        d.start()
                    if fuse:
                        x_desc(ci + 1, 1 - slot).start()

                @pl.when(ci >= 2)
                def _():
                    o_desc(ci - 2, slot).wait()

                for d in g_descs(ci, slot):
                    d.wait()
                if fuse:
                    x_desc(ci, slot).wait()
                gb = gath.at[slot]
                ob = outv.at[slot]
                if fuse:
                    xb = x0v.at[slot]

                def rowpair(rr, _):
                    vv = valv[pl.ds(ci * E + rr * 16, 16)]
                    for half in range(2):
                        r = rr * 2 + half
                        base = r * DEG
                        vals = [vv[half * DEG + d] for d in range(DEG)]
                        for cc in range(NCC):
                            cs = pl.ds(cc * 16, 16)
                            acc = vals[0] * gb[base, cs]
                            for d in range(1, DEG):
                                acc = acc + vals[d] * gb[base + d, cs]
                            if fuse:
                                acc = 2.0 * acc - xb[r, cs]
                            ob[r, cs] = acc
                    return 0

                lax.fori_loop(0, R // 2, rowpair, 0)
                o_desc(ci, slot).start()

            def loop_j(j, _):
                step(j * 2, 0)
                step(j * 2 + 1, 1)
                return 0

            lax.fori_loop(0, nch // 2, loop_j, 0)
            o_desc(nch - 2, 0).wait()
            o_desc(nch - 1, 1).wait()
            plsc.subcore_barrier()
            return 0

        lax.fori_loop(0, npass, run_pass, 0)

    kparams = pltpu.CompilerParams(use_tc_tiling_on_sc=False)
    if fuse:
        def run(X, col, val, Xprev):
            return pl.kernel(
                body,
                out_type=jax.ShapeDtypeStruct((V, C), jnp.float32),
                mesh=mesh,
                scratch_types=scratch,
                compiler_params=kparams,
            )(X, col, val, Xprev)
    else:
        def run(X, col, val):
            return pl.kernel(
                body,
                out_type=jax.ShapeDtypeStruct((V, C), jnp.float32),
                mesh=mesh,
                scratch_types=scratch,
                compiler_params=kparams,
            )(X, col, val)
    return run


def _linear1(x0, x1, x2, w0, w1, w2, b, blk=2048):
    V, C = x0.shape
    O = w0.shape[1]

    def body(x0_ref, x1_ref, x2_ref, w0_ref, w1_ref, w2_ref, b_ref, o_ref):
        y = lax.dot_general(x0_ref[...], w0_ref[...], (((1,), (0,)), ((), ())),
                            precision=_HIGH, preferred_element_type=jnp.float32)
        y = y + lax.dot_general(x1_ref[...], w1_ref[...], (((1,), (0,)), ((), ())),
                                precision=_HIGH, preferred_element_type=jnp.float32)
        y = y + lax.dot_general(x2_ref[...], w2_ref[...], (((1,), (0,)), ((), ())),
                                precision=_HIGH, preferred_element_type=jnp.float32)
        o_ref[...] = y + b_ref[...]

    wspec = pl.BlockSpec((C, O), lambda i: (0, 0))
    return pl.pallas_call(
        body,
        grid=(V // blk,),
        in_specs=[
            pl.BlockSpec((blk, C), lambda i: (i, 0)),
            pl.BlockSpec((blk, C), lambda i: (i, 0)),
            pl.BlockSpec((blk, C), lambda i: (i, 0)),
            wspec, wspec, wspec,
            pl.BlockSpec((1, O), lambda i: (0, 0)),
        ],
        out_specs=pl.BlockSpec((blk, O), lambda i: (i, 0)),
        out_shape=jax.ShapeDtypeStruct((V, O), jnp.float32),
    )(x0, x1, x2, w0, w1, w2, b)


def _linear3(x0, x1, x2, w0, w1, w2, b, F, O, pool, blk=4096):
    M = x0.shape[0]
    grid = M // blk
    oblk = blk // 4 if pool else blk

    def body(x0_ref, x1_ref, x2_ref, w0_ref, w1_ref, w2_ref, b_ref, o_ref):
        if F < 8:
            y = b_ref[...]
            for f in range(F):
                y = (y + x0_ref[:, f:f + 1] * w0_ref[f:f + 1, :]
                     + x1_ref[:, f:f + 1] * w1_ref[f:f + 1, :]
                     + x2_ref[:, f:f + 1] * w2_ref[f:f + 1, :])
        else:
            y = lax.dot_general(x0_ref[...], w0_ref[...], (((1,), (0,)), ((), ())),
                                precision=_HIGH, preferred_element_type=jnp.float32)
            y = y + lax.dot_general(x1_ref[...], w1_ref[...], (((1,), (0,)), ((), ())),
                                    precision=_HIGH, preferred_element_type=jnp.float32)
            y = y + lax.dot_general(x2_ref[...], w2_ref[...], (((1,), (0,)), ((), ())),
                                    precision=_HIGH, preferred_element_type=jnp.float32)
            y = y + b_ref[...]
        if pool:
            y = y.reshape(blk // (4 * B), 4, B, O).max(axis=1).reshape(oblk, O)
        o_ref[...] = y

    wspec = pl.BlockSpec((F, O), lambda i: (0, 0))
    return pl.pallas_call(
        body,
        grid=(grid,),
        in_specs=[
            pl.BlockSpec((blk, F), lambda i: (i, 0)),
            pl.BlockSpec((blk, F), lambda i: (i, 0)),
            pl.BlockSpec((blk, F), lambda i: (i, 0)),
            wspec, wspec, wspec,
            pl.BlockSpec((1, O), lambda i: (0, 0)),
        ],
        out_specs=pl.BlockSpec((oblk, O), lambda i: (i, 0)),
        out_shape=jax.ShapeDtypeStruct((M // 4 if pool else M, O), jnp.float32),
    )(x0, x1, x2, w0, w1, w2, b)


def _fc(Z2, Wfc1, bfc1, Wfc2, bfc2, ch=32):
    nv = Z2.shape[0] // B
    grid = nv // ch

    def body(z_ref, w1_ref, b1_ref, w2_ref, b2_ref, o_ref, acc_ref):
        i = pl.program_id(0)

        @pl.when(i == 0)
        def _():
            acc_ref[...] = jnp.zeros_like(acc_ref)

        acc = acc_ref[...]
        for j in range(ch):
            zz = z_ref[j * B:(j + 1) * B, :]
            wc = w1_ref[:, j * 64:(j + 1) * 64]
            acc = acc + lax.dot_general(zz, wc, (((1,), (1,)), ((), ())),
                                        precision=_HIGH,
                                        preferred_element_type=jnp.float32)
        acc_ref[...] = acc

        @pl.when(i == grid - 1)
        def _():
            h1 = acc_ref[...] + b1_ref[...]
            out = lax.dot_general(h1, w2_ref[...], (((1,), (1,)), ((), ())),
                                  precision=_HIGH,
                                  preferred_element_type=jnp.float32)
            o_ref[...] = out + b2_ref[...]

    return pl.pallas_call(
        body,
        grid=(grid,),
        in_specs=[
            pl.BlockSpec((ch * B, 64), lambda i: (i, 0)),
            pl.BlockSpec((512, ch * 64), lambda i: (0, i)),
            pl.BlockSpec((1, 512), lambda i: (0, 0)),
            pl.BlockSpec((NOUT, 512), lambda i: (0, 0)),
            pl.BlockSpec((1, NOUT), lambda i: (0, 0)),
        ],
        out_specs=pl.BlockSpec((B, NOUT), lambda i: (0, 0)),
        out_shape=jax.ShapeDtypeStruct((B, NOUT), jnp.float32),
        scratch_shapes=[pltpu.VMEM((B, 512), jnp.float32)],
    )(Z2, Wfc1, bfc1, Wfc2, bfc2)


def _wk(W, F, O):
    return [W[:, k::3].T for k in range(3)]


def kernel(x, W1, b1, W2, b2, W3, b3, W4, b4, Wfc1, bfc1, Wfc2, bfc2,
           L0_val, L1_val, L0_row, L0_col, L1_row, L1_col):
    del L0_row, L1_row

    spmm1 = _make_spmm(V0, B * FIN, 48, 64, False)
    spmm1f = _make_spmm(V0, B * FIN, 48, 64, True)
    spmm2 = _make_spmm(V0, B * 32, 32, 64, False)
    spmm2f = _make_spmm(V0, B * 32, 32, 64, True)
    spmm3 = _make_spmm(V1, B * 32, 128, 32, False)
    spmm3f = _make_spmm(V1, B * 32, 128, 32, True)
    spmm4 = _make_spmm(V1, B * 64, 128, 32, False)
    spmm4f = _make_spmm(V1, B * 64, 128, 32, True)

    X0 = jnp.transpose(x, (1, 0, 2)).reshape(V0, B * FIN)
    X1 = spmm1(X0, L0_col, L0_val)
    X2 = spmm1f(X1, L0_col, L0_val, X0)
    k0, k1, k2 = _wk(W1, FIN, 32)
    eyeB = jnp.eye(B, dtype=jnp.float32)
    H = _linear1(X0, X1, X2,
                 jnp.kron(eyeB, k0), jnp.kron(eyeB, k1), jnp.kron(eyeB, k2),
                 jnp.tile(b1, B).reshape(1, -1))
    H = H.reshape(V0, B * 32)

    X1 = spmm2(H, L0_col, L0_val)
    X2 = spmm2f(X1, L0_col, L0_val, H)
    k0, k1, k2 = _wk(W2, 32, 32)
    H = _linear3(H.reshape(-1, 32), X1.reshape(-1, 32), X2.reshape(-1, 32),
                 k0, k1, k2, b2.reshape(1, -1), 32, 32, True)
    H = H.reshape(V1, B * 32)

    X1 = spmm3(H, L1_col, L1_val)
    X2 = spmm3f(X1, L1_col, L1_val, H)
    k0, k1, k2 = _wk(W3, 32, 64)
    H = _linear3(H.reshape(-1, 32), X1.reshape(-1, 32), X2.reshape(-1, 32),
                 k0, k1, k2, b3.reshape(1, -1), 32, 64, False)
    H = H.reshape(V1, B * 64)

    X1 = spmm4(H, L1_col, L1_val)
    X2 = spmm4f(X1, L1_col, L1_val, H)
    k0, k1, k2 = _wk(W4, 64, 64)
    Z = _linear3(H.reshape(-1, 64), X1.reshape(-1, 64), X2.reshape(-1, 64),
                 k0, k1, k2, b4.reshape(1, -1), 64, 64, True)

    Z2 = Z.reshape(B * 1024, 64)
    return _fc(Z2, Wfc1, bfc1.reshape(1, -1), Wfc2, bfc2.reshape(1, -1))


# speedup vs baseline: 4.2059x; 1.1417x over previous
import functools

import jax
import jax.numpy as jnp
from jax import lax
from jax.experimental import pallas as pl
from jax.experimental.pallas import tpu as pltpu
from jax.experimental.pallas import tpu_sc as plsc

V0 = 16384
V1 = 4096
DEG = 8
B = 16
FIN = 3
NOUT = 72

_PREC = lax.Precision.HIGHEST


_SC_CORES = 2
_SC_SUBCORES = 16


def _make_spmm(V, C, CC, R, fuse):
    NS = _SC_SUBCORES
    P = C // CC
    split = P >= 2
    if split:
        assert P % 2 == 0
        rows_w = V // NS
    else:
        rows_w = V // (2 * NS)
    nch = rows_w // R
    assert nch % 2 == 0
    E = R * DEG
    NSUB = (E + 127) // 128
    assert E % NSUB == 0
    ES = E // NSUB
    RS = ES // DEG
    NCC = CC // 16
    EW = rows_w * DEG
    VT = V // NS

    mesh = plsc.VectorSubcoreMesh(core_axis_name="c", subcore_axis_name="s",
                                  num_cores=_SC_CORES, num_subcores=_SC_SUBCORES)
    scratch = [
        pltpu.VMEM((EW,), jnp.int32),
        pltpu.VMEM((EW,), jnp.float32),
        pltpu.VMEM_SHARED((V, CC), jnp.float32),
        pltpu.VMEM((2, E, CC), jnp.float32),
        pltpu.VMEM((2, R, CC), jnp.float32),
    ]
    if fuse:
        scratch.append(pltpu.VMEM((2, R, CC), jnp.float32))
    scratch.extend([pltpu.SemaphoreType.DMA] * (7 if fuse else 5))

    def body(*refs):
        if fuse:
            (x_hbm, col_hbm, val_hbm, x0_hbm, out_hbm,
             colv, valv, tab, gath, outv, x0v,
             g0, g1, o0, o1, xs0, xs1, ssem) = refs
            xsem = (xs0, xs1)
        else:
            (x_hbm, col_hbm, val_hbm, out_hbm,
             colv, valv, tab, gath, outv,
             g0, g1, o0, o1, ssem) = refs
        gsem = (g0, g1)
        osem = (o0, o1)
        sid = lax.axis_index("s")
        core = lax.axis_index("c")
        if split:
            w = sid
            npass = P // 2
        else:
            w = sid * _SC_CORES + core
            npass = P
        row0w = w * rows_w
        row0t = sid * VT

        pltpu.sync_copy(col_hbm.at[pl.ds(row0w * DEG, EW)], colv)
        pltpu.sync_copy(val_hbm.at[pl.ds(row0w * DEG, EW)], valv)

        def run_pass(j, _):
            p = j * 2 + core if split else j
            pc0 = p * CC

            pltpu.async_copy(
                x_hbm.at[pl.ds(row0t, VT), pl.ds(pc0, CC)],
                tab.at[pl.ds(row0t, VT)], ssem).wait()
            plsc.subcore_barrier()

            def x_desc(ci, slot):
                return pltpu.make_async_copy(
                    x0_hbm.at[pl.ds(row0w + ci * R, R), pl.ds(pc0, CC)],
                    x0v.at[slot], xsem[slot])

            def o_desc(ci, slot):
                return pltpu.make_async_copy(
                    outv.at[slot],
                    out_hbm.at[pl.ds(row0w + ci * R, R), pl.ds(pc0, CC)],
                    osem[slot])

            def g_descs(ci, slot):
                return [
                    pltpu.make_async_copy(
                        tab.at[colv.at[pl.ds(ci * E + s * ES, ES)]],
                        gath.at[slot].at[pl.ds(s * RS * DEG, ES)],
                        gsem[slot])
                    for s in range(NSUB)
                ]

            for d in g_descs(0, 0):
                d.start()
            if fuse:
                x_desc(0, 0).start()

            def step(ci, slot):
                @pl.when(ci + 1 < nch)
                def _():
                    for d in g_descs(ci + 1, 1 - slot):
                        d.start()
                    if fuse:
                        x_desc(ci + 1, 1 - slot).start()

                @pl.when(ci >= 2)
                def _():
                    o_desc(ci - 2, slot).wait()

                for d in g_descs(ci, slot):
                    d.wait()
                if fuse:
                    x_desc(ci, slot).wait()
                gb = gath.at[slot]
                ob = outv.at[slot]
                if fuse:
                    xb = x0v.at[slot]

                def rowpair(rr, _):
                    vv = valv[pl.ds(ci * E + rr * 16, 16)]
                    for half in range(2):
                        r = rr * 2 + half
                        base = r * DEG
                        vals = [vv[half * DEG + d] for d in range(DEG)]
                        for cc in range(NCC):
                            cs = pl.ds(cc * 16, 16)
                            acc = vals[0] * gb[base, cs]
                            for d in range(1, DEG):
                                acc = acc + vals[d] * gb[base + d, cs]
                            if fuse:
                                acc = 2.0 * acc - xb[r, cs]
                            ob[r, cs] = acc
                    return 0

                lax.fori_loop(0, R // 2, rowpair, 0)
                o_desc(ci, slot).start()

            def loop_j(j, _):
                step(j * 2, 0)
                step(j * 2 + 1, 1)
                return 0

            lax.fori_loop(0, nch // 2, loop_j, 0)
            o_desc(nch - 2, 0).wait()
            o_desc(nch - 1, 1).wait()
            plsc.subcore_barrier()
            return 0

        lax.fori_loop(0, npass, run_pass, 0)

    kparams = pltpu.CompilerParams(use_tc_tiling_on_sc=False)
    if fuse:
        def run(X, col, val, Xprev):
            return pl.kernel(
                body,
                out_type=jax.ShapeDtypeStruct((V, C), jnp.float32),
                mesh=mesh,
                scratch_types=scratch,
                compiler_params=kparams,
            )(X, col, val, Xprev)
    else:
        def run(X, col, val):
            return pl.kernel(
                body,
                out_type=jax.ShapeDtypeStruct((V, C), jnp.float32),
                mesh=mesh,
                scratch_types=scratch,
                compiler_params=kparams,
            )(X, col, val)
    return run


def _linear_vm(x0, x1, x2, m0, m1, m2, b, pool, blk):
    V, Cin = x0.shape
    Cout = m0.shape[1]
    oblk = blk // 4 if pool else blk

    def body(x0_ref, x1_ref, x2_ref, m0_ref, m1_ref, m2_ref, b_ref, o_ref):
        y = lax.dot_general(x0_ref[...], m0_ref[...], (((1,), (0,)), ((), ())),
                            precision=_PREC, preferred_element_type=jnp.float32)
        y = y + lax.dot_general(x1_ref[...], m1_ref[...], (((1,), (0,)), ((), ())),
                                precision=_PREC, preferred_element_type=jnp.float32)
        y = y + lax.dot_general(x2_ref[...], m2_ref[...], (((1,), (0,)), ((), ())),
                                precision=_PREC, preferred_element_type=jnp.float32)
        y = y + b_ref[...]
        if pool:
            y = y.reshape(blk // 4, 4, Cout).max(axis=1)
        o_ref[...] = y

    wspec = pl.BlockSpec((Cin, Cout), lambda i: (0, 0))
    return pl.pallas_call(
        body,
        grid=(V // blk,),
        in_specs=[
            pl.BlockSpec((blk, Cin), lambda i: (i, 0)),
            pl.BlockSpec((blk, Cin), lambda i: (i, 0)),
            pl.BlockSpec((blk, Cin), lambda i: (i, 0)),
            wspec, wspec, wspec,
            pl.BlockSpec((1, Cout), lambda i: (0, 0)),
        ],
        out_specs=pl.BlockSpec((oblk, Cout), lambda i: (i, 0)),
        out_shape=jax.ShapeDtypeStruct((V // 4 if pool else V, Cout),
                                       jnp.float32),
    )(x0, x1, x2, m0, m1, m2, b)


def _fc(Zt, Wfc1, bfc1, Wfc2, bfc2, ch=2048):
    K = Zt.shape[0]
    grid = K // ch

    def body(z_ref, w1_ref, b1_ref, w2_ref, b2_ref, o_ref, acc_ref):
        i = pl.program_id(0)

        @pl.when(i == 0)
        def _():
            acc_ref[...] = jnp.zeros_like(acc_ref)

        acc_ref[...] += lax.dot_general(
            w1_ref[...], z_ref[...], (((1,), (0,)), ((), ())),
            precision=_PREC, preferred_element_type=jnp.float32)

        @pl.when(i == grid - 1)
        def _():
            h1 = acc_ref[...] + b1_ref[...]
            out = lax.dot_general(w2_ref[...], h1, (((1,), (0,)), ((), ())),
                                  precision=_PREC,
                                  preferred_element_type=jnp.float32)
            o_ref[...] = out + b2_ref[...]

    return pl.pallas_call(
        body,
        grid=(grid,),
        in_specs=[
            pl.BlockSpec((ch, B), lambda i: (i, 0)),
            pl.BlockSpec((512, ch), lambda i: (0, i)),
            pl.BlockSpec((512, 1), lambda i: (0, 0)),
            pl.BlockSpec((NOUT, 512), lambda i: (0, 0)),
            pl.BlockSpec((NOUT, 1), lambda i: (0, 0)),
        ],
        out_specs=pl.BlockSpec((NOUT, B), lambda i: (0, 0)),
        out_shape=jax.ShapeDtypeStruct((NOUT, B), jnp.float32),
        scratch_shapes=[pltpu.VMEM((512, B), jnp.float32)],
    )(Zt, Wfc1, bfc1, Wfc2, bfc2)


def _wk(W):
    return [W[:, k::3].T for k in range(3)]


def kernel(x, W1, b1, W2, b2, W3, b3, W4, b4, Wfc1, bfc1, Wfc2, bfc2,
           L0_val, L1_val, L0_row, L0_col, L1_row, L1_col):
    del L0_row, L1_row

    spmm1 = _make_spmm(V0, B * FIN, 48, 64, False)
    spmm1f = _make_spmm(V0, B * FIN, 48, 64, True)
    spmm2 = _make_spmm(V0, B * 32, 32, 64, False)
    spmm2f = _make_spmm(V0, B * 32, 32, 64, True)
    spmm3 = _make_spmm(V1, B * 32, 128, 32, False)
    spmm3f = _make_spmm(V1, B * 32, 128, 32, True)
    spmm4 = _make_spmm(V1, B * 64, 128, 32, False)
    spmm4f = _make_spmm(V1, B * 64, 128, 32, True)

    eyeB = jnp.eye(B, dtype=jnp.float32)

    def lift_bm(wk):
        return jnp.kron(eyeB, wk)

    def lift_om(wk):
        F, O = wk.shape
        return jnp.einsum('fo,bc->bfoc', wk, eyeB).reshape(B * F, O * B)

    X0 = jnp.transpose(x, (1, 0, 2)).reshape(V0, B * FIN)
    X1 = spmm1(X0, L0_col, L0_val)
    X2 = spmm1f(X1, L0_col, L0_val, X0)
    k0, k1, k2 = _wk(W1)
    H = _linear_vm(X0, X1, X2, lift_bm(k0), lift_bm(k1), lift_bm(k2),
                   jnp.tile(b1, B).reshape(1, -1), False, 2048)

    X1 = spmm2(H, L0_col, L0_val)
    X2 = spmm2f(X1, L0_col, L0_val, H)
    k0, k1, k2 = _wk(W2)
    H = _linear_vm(H, X1, X2, lift_bm(k0), lift_bm(k1), lift_bm(k2),
                   jnp.tile(b2, B).reshape(1, -1), True, 1024)

    X1 = spmm3(H, L1_col, L1_val)
    X2 = spmm3f(X1, L1_col, L1_val, H)
    k0, k1, k2 = _wk(W3)
    H = _linear_vm(H, X1, X2, lift_bm(k0), lift_bm(k1), lift_bm(k2),
                   jnp.tile(b3, B).reshape(1, -1), False, 1024)

    X1 = spmm4(H, L1_col, L1_val)
    X2 = spmm4f(X1, L1_col, L1_val, H)
    k0, k1, k2 = _wk(W4)
    Z = _linear_vm(H, X1, X2, lift_om(k0), lift_om(k1), lift_om(k2),
                   jnp.repeat(b4, B).reshape(1, -1), True, 512)

    Zt = Z.reshape(B * 1024 * 4, B)
    outT = _fc(Zt, Wfc1, bfc1.reshape(-1, 1), Wfc2, bfc2.reshape(-1, 1))
    return outT.T


# speedup vs baseline: 5.4362x; 1.2925x over previous
import functools

import jax
import jax.numpy as jnp
from jax import lax
from jax.experimental import pallas as pl
from jax.experimental.pallas import tpu as pltpu
from jax.experimental.pallas import tpu_sc as plsc

V0 = 16384
V1 = 4096
DEG = 8
B = 16
FIN = 3
NOUT = 72

_PREC = lax.Precision.DEFAULT


_SC_CORES = 2
_SC_SUBCORES = 16


def _make_spmm(V, C, CC, R, fuse):
    NS = _SC_SUBCORES
    P = C // CC
    split = P >= 2
    if split:
        assert P % 2 == 0
        rows_w = V // NS
    else:
        rows_w = V // (2 * NS)
    nch = rows_w // R
    assert nch % 2 == 0
    E = R * DEG
    NSUB = (E + 127) // 128
    assert E % NSUB == 0
    ES = E // NSUB
    RS = ES // DEG
    NCC = CC // 16
    EW = rows_w * DEG
    VT = V // NS

    mesh = plsc.VectorSubcoreMesh(core_axis_name="c", subcore_axis_name="s",
                                  num_cores=_SC_CORES, num_subcores=_SC_SUBCORES)
    scratch = [
        pltpu.VMEM((EW,), jnp.int32),
        pltpu.VMEM((EW,), jnp.float32),
        pltpu.VMEM_SHARED((V, CC), jnp.float32),
        pltpu.VMEM((2, E, CC), jnp.float32),
        pltpu.VMEM((2, R, CC), jnp.float32),
    ]
    if fuse:
        scratch.append(pltpu.VMEM((2, R, CC), jnp.float32))
    scratch.extend([pltpu.SemaphoreType.DMA] * (7 if fuse else 5))

    def body(*refs):
        if fuse:
            (x_hbm, col_hbm, val_hbm, x0_hbm, out_hbm,
             colv, valv, tab, gath, outv, x0v,
             g0, g1, o0, o1, xs0, xs1, ssem) = refs
            xsem = (xs0, xs1)
        else:
            (x_hbm, col_hbm, val_hbm, out_hbm,
             colv, valv, tab, gath, outv,
             g0, g1, o0, o1, ssem) = refs
        gsem = (g0, g1)
        osem = (o0, o1)
        sid = lax.axis_index("s")
        core = lax.axis_index("c")
        if split:
            w = sid
            npass = P // 2
        else:
            w = sid * _SC_CORES + core
            npass = P
        row0w = w * rows_w
        row0t = sid * VT

        pltpu.sync_copy(col_hbm.at[pl.ds(row0w * DEG, EW)], colv)
        pltpu.sync_copy(val_hbm.at[pl.ds(row0w * DEG, EW)], valv)

        def run_pass(j, _):
            p = j * 2 + core if split else j
            pc0 = p * CC

            pltpu.async_copy(
                x_hbm.at[pl.ds(row0t, VT), pl.ds(pc0, CC)],
                tab.at[pl.ds(row0t, VT)], ssem).wait()
            plsc.subcore_barrier()

            def x_desc(ci, slot):
                return pltpu.make_async_copy(
                    x0_hbm.at[pl.ds(row0w + ci * R, R), pl.ds(pc0, CC)],
                    x0v.at[slot], xsem[slot])

            def o_desc(ci, slot):
                return pltpu.make_async_copy(
                    outv.at[slot],
                    out_hbm.at[pl.ds(row0w + ci * R, R), pl.ds(pc0, CC)],
                    osem[slot])

            def g_descs(ci, slot):
                return [
                    pltpu.make_async_copy(
                        tab.at[colv.at[pl.ds(ci * E + s * ES, ES)]],
                        gath.at[slot].at[pl.ds(s * RS * DEG, ES)],
                        gsem[slot])
                    for s in range(NSUB)
                ]

            for d in g_descs(0, 0):
                d.start()
            if fuse:
                x_desc(0, 0).start()

            def step(ci, slot):
                @pl.when(ci + 1 < nch)
                def _():
                    for d in g_descs(ci + 1, 1 - slot):
                        d.start()
                    if fuse:
                        x_desc(ci + 1, 1 - slot).start()

                @pl.when(ci >= 2)
                def _():
                    o_desc(ci - 2, slot).wait()

                for d in g_descs(ci, slot):
                    d.wait()
                if fuse:
                    x_desc(ci, slot).wait()
                gb = gath.at[slot]
                ob = outv.at[slot]
                if fuse:
                    xb = x0v.at[slot]

                def rowpair(rr, _):
                    vv = valv[pl.ds(ci * E + rr * 16, 16)]
                    for half in range(2):
                        r = rr * 2 + half
                        base = r * DEG
                        vals = [vv[half * DEG + d] for d in range(DEG)]
                        for cc in range(NCC):
                            cs = pl.ds(cc * 16, 16)
                            acc = vals[0] * gb[base, cs]
                            for d in range(1, DEG):
                                acc = acc + vals[d] * gb[base + d, cs]
                            if fuse:
                                acc = 2.0 * acc - xb[r, cs]
                            ob[r, cs] = acc
                    return 0

                lax.fori_loop(0, R // 2, rowpair, 0)
                o_desc(ci, slot).start()

            def loop_j(j, _):
                step(j * 2, 0)
                step(j * 2 + 1, 1)
                return 0

            lax.fori_loop(0, nch // 2, loop_j, 0)
            o_desc(nch - 2, 0).wait()
            o_desc(nch - 1, 1).wait()
            plsc.subcore_barrier()
            return 0

        lax.fori_loop(0, npass, run_pass, 0)

    kparams = pltpu.CompilerParams(use_tc_tiling_on_sc=False)
    if fuse:
        def run(X, col, val, Xprev):
            return pl.kernel(
                body,
                out_type=jax.ShapeDtypeStruct((V, C), jnp.float32),
                mesh=mesh,
                scratch_types=scratch,
                compiler_params=kparams,
            )(X, col, val, Xprev)
    else:
        def run(X, col, val):
            return pl.kernel(
                body,
                out_type=jax.ShapeDtypeStruct((V, C), jnp.float32),
                mesh=mesh,
                scratch_types=scratch,
                compiler_params=kparams,
            )(X, col, val)
    return run


def _linear_vm(x0, x1, x2, m0, m1, m2, b, pool, blk):
    V, Cin = x0.shape
    Cout = m0.shape[1]
    oblk = blk // 4 if pool else blk

    def body(x0_ref, x1_ref, x2_ref, m0_ref, m1_ref, m2_ref, b_ref, o_ref):
        y = lax.dot_general(x0_ref[...], m0_ref[...], (((1,), (0,)), ((), ())),
                            precision=_PREC, preferred_element_type=jnp.float32)
        y = y + lax.dot_general(x1_ref[...], m1_ref[...], (((1,), (0,)), ((), ())),
                                precision=_PREC, preferred_element_type=jnp.float32)
        y = y + lax.dot_general(x2_ref[...], m2_ref[...], (((1,), (0,)), ((), ())),
                                precision=_PREC, preferred_element_type=jnp.float32)
        y = y + b_ref[...]
        if pool:
            y = y.reshape(blk // 4, 4, Cout).max(axis=1)
        o_ref[...] = y

    wspec = pl.BlockSpec((Cin, Cout), lambda i: (0, 0))
    return pl.pallas_call(
        body,
        grid=(V // blk,),
        in_specs=[
            pl.BlockSpec((blk, Cin), lambda i: (i, 0)),
            pl.BlockSpec((blk, Cin), lambda i: (i, 0)),
            pl.BlockSpec((blk, Cin), lambda i: (i, 0)),
            wspec, wspec, wspec,
            pl.BlockSpec((1, Cout), lambda i: (0, 0)),
        ],
        out_specs=pl.BlockSpec((oblk, Cout), lambda i: (i, 0)),
        out_shape=jax.ShapeDtypeStruct((V // 4 if pool else V, Cout),
                                       jnp.float32),
    )(x0, x1, x2, m0, m1, m2, b)


def _fc(Zt, Wfc1, bfc1, Wfc2, bfc2, ch=2048):
    K = Zt.shape[0]
    grid = K // ch

    def body(z_ref, w1_ref, b1_ref, w2_ref, b2_ref, o_ref, acc_ref):
        i = pl.program_id(0)

        @pl.when(i == 0)
        def _():
            acc_ref[...] = jnp.zeros_like(acc_ref)

        acc_ref[...] += lax.dot_general(
            w1_ref[...], z_ref[...], (((1,), (0,)), ((), ())),
            precision=_PREC, preferred_element_type=jnp.float32)

        @pl.when(i == grid - 1)
        def _():
            h1 = acc_ref[...] + b1_ref[...]
            out = lax.dot_general(w2_ref[...], h1, (((1,), (0,)), ((), ())),
                                  precision=_PREC,
                                  preferred_element_type=jnp.float32)
            o_ref[...] = out + b2_ref[...]

    return pl.pallas_call(
        body,
        grid=(grid,),
        in_specs=[
            pl.BlockSpec((ch, B), lambda i: (i, 0)),
            pl.BlockSpec((512, ch), lambda i: (0, i)),
            pl.BlockSpec((512, 1), lambda i: (0, 0)),
            pl.BlockSpec((NOUT, 512), lambda i: (0, 0)),
            pl.BlockSpec((NOUT, 1), lambda i: (0, 0)),
        ],
        out_specs=pl.BlockSpec((NOUT, B), lambda i: (0, 0)),
        out_shape=jax.ShapeDtypeStruct((NOUT, B), jnp.float32),
        scratch_shapes=[pltpu.VMEM((512, B), jnp.float32)],
    )(Zt, Wfc1, bfc1, Wfc2, bfc2)


def _wk(W):
    return [W[:, k::3].T for k in range(3)]


def kernel(x, W1, b1, W2, b2, W3, b3, W4, b4, Wfc1, bfc1, Wfc2, bfc2,
           L0_val, L1_val, L0_row, L0_col, L1_row, L1_col):
    del L0_row, L1_row

    spmm1 = _make_spmm(V0, B * FIN, 48, 64, False)
    spmm1f = _make_spmm(V0, B * FIN, 48, 64, True)
    spmm2 = _make_spmm(V0, B * 32, 32, 64, False)
    spmm2f = _make_spmm(V0, B * 32, 32, 64, True)
    spmm3 = _make_spmm(V1, B * 32, 128, 32, False)
    spmm3f = _make_spmm(V1, B * 32, 128, 32, True)
    spmm4 = _make_spmm(V1, B * 64, 128, 32, False)
    spmm4f = _make_spmm(V1, B * 64, 128, 32, True)

    eyeB = jnp.eye(B, dtype=jnp.float32)

    def lift_bm(wk):
        return jnp.kron(eyeB, wk)

    def lift_om(wk):
        F, O = wk.shape
        return jnp.einsum('fo,bc->bfoc', wk, eyeB).reshape(B * F, O * B)

    X0 = jnp.transpose(x, (1, 0, 2)).reshape(V0, B * FIN)
    X1 = spmm1(X0, L0_col, L0_val)
    X2 = spmm1f(X1, L0_col, L0_val, X0)
    k0, k1, k2 = _wk(W1)
    H = _linear_vm(X0, X1, X2, lift_bm(k0), lift_bm(k1), lift_bm(k2),
                   jnp.tile(b1, B).reshape(1, -1), False, 2048)

    X1 = spmm2(H, L0_col, L0_val)
    X2 = spmm2f(X1, L0_col, L0_val, H)
    k0, k1, k2 = _wk(W2)
    H = _linear_vm(H, X1, X2, lift_bm(k0), lift_bm(k1), lift_bm(k2),
                   jnp.tile(b2, B).reshape(1, -1), True, 1024)

    X1 = spmm3(H, L1_col, L1_val)
    X2 = spmm3f(X1, L1_col, L1_val, H)
    k0, k1, k2 = _wk(W3)
    H = _linear_vm(H, X1, X2, lift_bm(k0), lift_bm(k1), lift_bm(k2),
                   jnp.tile(b3, B).reshape(1, -1), False, 1024)

    X1 = spmm4(H, L1_col, L1_val)
    X2 = spmm4f(X1, L1_col, L1_val, H)
    k0, k1, k2 = _wk(W4)
    Z = _linear_vm(H, X1, X2, lift_om(k0), lift_om(k1), lift_om(k2),
                   jnp.repeat(b4, B).reshape(1, -1), True, 512)

    Zt = Z.reshape(B * 1024 * 4, B)
    outT = _fc(Zt, Wfc1, bfc1.reshape(-1, 1), Wfc2, bfc2.reshape(-1, 1))
    return outT.T


# speedup vs baseline: 6.7792x; 1.2470x over previous
import functools

import jax
import jax.numpy as jnp
from jax import lax
from jax.experimental import pallas as pl
from jax.experimental.pallas import tpu as pltpu
from jax.experimental.pallas import tpu_sc as plsc

V0 = 16384
V1 = 4096
DEG = 8
B = 16
FIN = 3
NOUT = 72

_PREC = lax.Precision.DEFAULT


_SC_CORES = 2
_SC_SUBCORES = 16


def _make_spmm(V, C, CC, R, fuse, adt=jnp.float32):
    bf = adt == jnp.bfloat16
    NS = _SC_SUBCORES
    P = C // CC
    split = P >= 2
    if split:
        assert P % 2 == 0
        rows_w = V // NS
    else:
        rows_w = V // (2 * NS)
    nch = rows_w // R
    assert nch % 2 == 0
    E = R * DEG
    NSUB = (E + 127) // 128
    assert E % NSUB == 0
    ES = E // NSUB
    RS = ES // DEG
    NCC = CC // 16
    EW = rows_w * DEG
    VT = V // NS

    mesh = plsc.VectorSubcoreMesh(core_axis_name="c", subcore_axis_name="s",
                                  num_cores=_SC_CORES, num_subcores=_SC_SUBCORES)
    scratch = [
        pltpu.VMEM((EW,), jnp.int32),
        pltpu.VMEM((EW,), jnp.float32),
        pltpu.VMEM_SHARED((V, CC), adt),
        pltpu.VMEM((2, E, CC), adt),
        pltpu.VMEM((2, R, CC), adt),
    ]
    if fuse:
        scratch.append(pltpu.VMEM((2, R, CC), adt))
    scratch.extend([pltpu.SemaphoreType.DMA] * (7 if fuse else 5))

    def body(*refs):
        if fuse:
            (x_hbm, col_hbm, val_hbm, x0_hbm, out_hbm,
             colv, valv, tab, gath, outv, x0v,
             g0, g1, o0, o1, xs0, xs1, ssem) = refs
            xsem = (xs0, xs1)
        else:
            (x_hbm, col_hbm, val_hbm, out_hbm,
             colv, valv, tab, gath, outv,
             g0, g1, o0, o1, ssem) = refs
        gsem = (g0, g1)
        osem = (o0, o1)
        sid = lax.axis_index("s")
        core = lax.axis_index("c")
        if split:
            w = sid
            npass = P // 2
        else:
            w = sid * _SC_CORES + core
            npass = P
        row0w = w * rows_w
        row0t = sid * VT

        pltpu.sync_copy(col_hbm.at[pl.ds(row0w * DEG, EW)], colv)
        pltpu.sync_copy(val_hbm.at[pl.ds(row0w * DEG, EW)], valv)

        def run_pass(j, _):
            p = j * 2 + core if split else j
            pc0 = p * CC

            pltpu.async_copy(
                x_hbm.at[pl.ds(row0t, VT), pl.ds(pc0, CC)],
                tab.at[pl.ds(row0t, VT)], ssem).wait()
            plsc.subcore_barrier()

            def x_desc(ci, slot):
                return pltpu.make_async_copy(
                    x0_hbm.at[pl.ds(row0w + ci * R, R), pl.ds(pc0, CC)],
                    x0v.at[slot], xsem[slot])

            def o_desc(ci, slot):
                return pltpu.make_async_copy(
                    outv.at[slot],
                    out_hbm.at[pl.ds(row0w + ci * R, R), pl.ds(pc0, CC)],
                    osem[slot])

            def g_descs(ci, slot):
                return [
                    pltpu.make_async_copy(
                        tab.at[colv.at[pl.ds(ci * E + s * ES, ES)]],
                        gath.at[slot].at[pl.ds(s * RS * DEG, ES)],
                        gsem[slot])
                    for s in range(NSUB)
                ]

            for d in g_descs(0, 0):
                d.start()
            if fuse:
                x_desc(0, 0).start()

            def step(ci, slot):
                @pl.when(ci + 1 < nch)
                def _():
                    for d in g_descs(ci + 1, 1 - slot):
                        d.start()
                    if fuse:
                        x_desc(ci + 1, 1 - slot).start()

                @pl.when(ci >= 2)
                def _():
                    o_desc(ci - 2, slot).wait()

                for d in g_descs(ci, slot):
                    d.wait()
                if fuse:
                    x_desc(ci, slot).wait()
                gb = gath.at[slot]
                ob = outv.at[slot]
                if fuse:
                    xb = x0v.at[slot]

                def rowpair(rr, _):
                    vv = valv[pl.ds(ci * E + rr * 16, 16)]
                    for half in range(2):
                        r = rr * 2 + half
                        base = r * DEG
                        vals = [vv[half * DEG + d] for d in range(DEG)]
                        if bf:
                            for cc in range(CC // 32):
                                cs = pl.ds(cc * 32, 32)
                                ga, go = plsc.unpack(
                                    gb[base, cs],
                                    format=plsc.PackFormat.INTERLEAVED)
                                aa = vals[0] * ga
                                ao = vals[0] * go
                                for d in range(1, DEG):
                                    ga, go = plsc.unpack(
                                        gb[base + d, cs],
                                        format=plsc.PackFormat.INTERLEAVED)
                                    aa = aa + vals[d] * ga
                                    ao = ao + vals[d] * go
                                if fuse:
                                    xa, xo = plsc.unpack(
                                        xb[r, cs],
                                        format=plsc.PackFormat.INTERLEAVED)
                                    aa = 2.0 * aa - xa
                                    ao = 2.0 * ao - xo
                                ob[r, cs] = plsc.pack(
                                    aa, ao, format=plsc.PackFormat.INTERLEAVED)
                        else:
                            for cc in range(NCC):
                                cs = pl.ds(cc * 16, 16)
                                acc = vals[0] * gb[base, cs]
                                for d in range(1, DEG):
                                    acc = acc + vals[d] * gb[base + d, cs]
                                if fuse:
                                    acc = 2.0 * acc - xb[r, cs]
                                ob[r, cs] = acc
                    return 0

                lax.fori_loop(0, R // 2, rowpair, 0)
                o_desc(ci, slot).start()

            def loop_j(j, _):
                step(j * 2, 0)
                step(j * 2 + 1, 1)
                return 0

            lax.fori_loop(0, nch // 2, loop_j, 0)
            o_desc(nch - 2, 0).wait()
            o_desc(nch - 1, 1).wait()
            plsc.subcore_barrier()
            return 0

        lax.fori_loop(0, npass, run_pass, 0)

    kparams = pltpu.CompilerParams(use_tc_tiling_on_sc=False,
                                   needs_layout_passes=not bf)
    if fuse:
        def run(X, col, val, Xprev):
            return pl.kernel(
                body,
                out_type=jax.ShapeDtypeStruct((V, C), adt),
                mesh=mesh,
                scratch_types=scratch,
                compiler_params=kparams,
            )(X, col, val, Xprev)
    else:
        def run(X, col, val):
            return pl.kernel(
                body,
                out_type=jax.ShapeDtypeStruct((V, C), adt),
                mesh=mesh,
                scratch_types=scratch,
                compiler_params=kparams,
            )(X, col, val)
    return run


def _linear_vm(x0, x1, x2, m0, m1, m2, b, pool, blk, odt=jnp.float32):
    V, Cin = x0.shape
    Cout = m0.shape[1]
    oblk = blk // 4 if pool else blk

    def body(x0_ref, x1_ref, x2_ref, m0_ref, m1_ref, m2_ref, b_ref, o_ref):
        xs = [x0_ref[...].astype(jnp.float32), x1_ref[...].astype(jnp.float32),
              x2_ref[...].astype(jnp.float32)]
        y = lax.dot_general(xs[0], m0_ref[...], (((1,), (0,)), ((), ())),
                            precision=_PREC, preferred_element_type=jnp.float32)
        y = y + lax.dot_general(xs[1], m1_ref[...], (((1,), (0,)), ((), ())),
                                precision=_PREC, preferred_element_type=jnp.float32)
        y = y + lax.dot_general(xs[2], m2_ref[...], (((1,), (0,)), ((), ())),
                                precision=_PREC, preferred_element_type=jnp.float32)
        y = y + b_ref[...]
        if pool:
            y = y.reshape(blk // 4, 4, Cout).max(axis=1)
        o_ref[...] = y.astype(odt)

    wspec = pl.BlockSpec((Cin, Cout), lambda i: (0, 0))
    return pl.pallas_call(
        body,
        grid=(V // blk,),
        in_specs=[
            pl.BlockSpec((blk, Cin), lambda i: (i, 0)),
            pl.BlockSpec((blk, Cin), lambda i: (i, 0)),
            pl.BlockSpec((blk, Cin), lambda i: (i, 0)),
            wspec, wspec, wspec,
            pl.BlockSpec((1, Cout), lambda i: (0, 0)),
        ],
        out_specs=pl.BlockSpec((oblk, Cout), lambda i: (i, 0)),
        out_shape=jax.ShapeDtypeStruct((V // 4 if pool else V, Cout), odt),
    )(x0, x1, x2, m0, m1, m2, b)


def _fc(Zt, Wfc1, bfc1, Wfc2, bfc2, ch=2048):
    K = Zt.shape[0]
    grid = K // ch

    def body(z_ref, w1_ref, b1_ref, w2_ref, b2_ref, o_ref, acc_ref):
        i = pl.program_id(0)

        @pl.when(i == 0)
        def _():
            acc_ref[...] = jnp.zeros_like(acc_ref)

        acc_ref[...] += lax.dot_general(
            w1_ref[...], z_ref[...], (((1,), (0,)), ((), ())),
            precision=_PREC, preferred_element_type=jnp.float32)

        @pl.when(i == grid - 1)
        def _():
            h1 = acc_ref[...] + b1_ref[...]
            out = lax.dot_general(w2_ref[...], h1, (((1,), (0,)), ((), ())),
                                  precision=_PREC,
                                  preferred_element_type=jnp.float32)
            o_ref[...] = out + b2_ref[...]

    return pl.pallas_call(
        body,
        grid=(grid,),
        in_specs=[
            pl.BlockSpec((ch, B), lambda i: (i, 0)),
            pl.BlockSpec((512, ch), lambda i: (0, i)),
            pl.BlockSpec((512, 1), lambda i: (0, 0)),
            pl.BlockSpec((NOUT, 512), lambda i: (0, 0)),
            pl.BlockSpec((NOUT, 1), lambda i: (0, 0)),
        ],
        out_specs=pl.BlockSpec((NOUT, B), lambda i: (0, 0)),
        out_shape=jax.ShapeDtypeStruct((NOUT, B), jnp.float32),
        scratch_shapes=[pltpu.VMEM((512, B), jnp.float32)],
    )(Zt, Wfc1, bfc1, Wfc2, bfc2)


def _wk(W):
    return [W[:, k::3].T for k in range(3)]


def kernel(x, W1, b1, W2, b2, W3, b3, W4, b4, Wfc1, bfc1, Wfc2, bfc2,
           L0_val, L1_val, L0_row, L0_col, L1_row, L1_col):
    del L0_row, L1_row

    spmm1 = _make_spmm(V0, B * FIN, 48, 64, False)
    spmm1f = _make_spmm(V0, B * FIN, 48, 64, True)
    bf16 = jnp.bfloat16
    spmm2 = _make_spmm(V0, B * 32, 64, 64, False, bf16)
    spmm2f = _make_spmm(V0, B * 32, 64, 64, True, bf16)
    spmm3 = _make_spmm(V1, B * 32, 256, 32, False, bf16)
    spmm3f = _make_spmm(V1, B * 32, 256, 32, True, bf16)
    spmm4 = _make_spmm(V1, B * 64, 256, 32, False, bf16)
    spmm4f = _make_spmm(V1, B * 64, 256, 32, True, bf16)

    eyeB = jnp.eye(B, dtype=jnp.float32)

    def lift_bm(wk):
        return jnp.kron(eyeB, wk)

    def lift_om(wk):
        F, O = wk.shape
        return jnp.einsum('fo,bc->bfoc', wk, eyeB).reshape(B * F, O * B)

    X0 = jnp.transpose(x, (1, 0, 2)).reshape(V0, B * FIN)
    X1 = spmm1(X0, L0_col, L0_val)
    X2 = spmm1f(X1, L0_col, L0_val, X0)
    k0, k1, k2 = _wk(W1)
    H = _linear_vm(X0, X1, X2, lift_bm(k0), lift_bm(k1), lift_bm(k2),
                   jnp.tile(b1, B).reshape(1, -1), False, 2048, jnp.bfloat16)

    X1 = spmm2(H, L0_col, L0_val)
    X2 = spmm2f(X1, L0_col, L0_val, H)
    k0, k1, k2 = _wk(W2)
    H = _linear_vm(H, X1, X2, lift_bm(k0), lift_bm(k1), lift_bm(k2),
                   jnp.tile(b2, B).reshape(1, -1), True, 1024, jnp.bfloat16)

    X1 = spmm3(H, L1_col, L1_val)
    X2 = spmm3f(X1, L1_col, L1_val, H)
    k0, k1, k2 = _wk(W3)
    H = _linear_vm(H, X1, X2, lift_bm(k0), lift_bm(k1), lift_bm(k2),
                   jnp.tile(b3, B).reshape(1, -1), False, 1024, jnp.bfloat16)

    X1 = spmm4(H, L1_col, L1_val)
    X2 = spmm4f(X1, L1_col, L1_val, H)
    k0, k1, k2 = _wk(W4)
    Z = _linear_vm(H, X1, X2, lift_om(k0), lift_om(k1), lift_om(k2),
                   jnp.repeat(b4, B).reshape(1, -1), True, 512)

    Zt = Z.reshape(B * 1024 * 4, B)
    outT = _fc(Zt, Wfc1, bfc1.reshape(-1, 1), Wfc2, bfc2.reshape(-1, 1))
    return outT.T


# speedup vs baseline: 6.7815x; 1.0003x over previous
import jax
import jax.numpy as jnp
from jax import lax
from jax.experimental import pallas as pl
from jax.experimental.pallas import tpu as pltpu
from jax.experimental.pallas import tpu_sc as plsc

V0 = 16384
V1 = 4096
DEG = 8
B = 16
FIN = 3
NOUT = 72

_PREC = lax.Precision.DEFAULT


_SC_CORES = 2
_SC_SUBCORES = 16


def _make_spmm(V, C, CC, R, fuse, adt=jnp.float32):
    bf = adt == jnp.bfloat16
    NS = _SC_SUBCORES
    P = C // CC
    split = P >= 2
    if split:
        assert P % 2 == 0
        rows_w = V // NS
    else:
        rows_w = V // (2 * NS)
    nch = rows_w // R
    assert nch % 2 == 0
    E = R * DEG
    NSUB = (E + 127) // 128
    assert E % NSUB == 0
    ES = E // NSUB
    RS = ES // DEG
    NCC = CC // 16
    EW = rows_w * DEG
    VT = V // NS

    mesh = plsc.VectorSubcoreMesh(core_axis_name="c", subcore_axis_name="s",
                                  num_cores=_SC_CORES, num_subcores=_SC_SUBCORES)
    scratch = [
        pltpu.VMEM((EW,), jnp.int32),
        pltpu.VMEM((EW,), jnp.float32),
        pltpu.VMEM_SHARED((V, CC), adt),
        pltpu.VMEM((2, E, CC), adt),
        pltpu.VMEM((2, R, CC), adt),
    ]
    if fuse:
        scratch.append(pltpu.VMEM((2, R, CC), adt))
    scratch.extend([pltpu.SemaphoreType.DMA] * (7 if fuse else 5))

    def body(*refs):
        if fuse:
            (x_hbm, col_hbm, val_hbm, x0_hbm, out_hbm,
             colv, valv, tab, gath, outv, x0v,
             g0, g1, o0, o1, xs0, xs1, ssem) = refs
            xsem = (xs0, xs1)
        else:
            (x_hbm, col_hbm, val_hbm, out_hbm,
             colv, valv, tab, gath, outv,
             g0, g1, o0, o1, ssem) = refs
        gsem = (g0, g1)
        osem = (o0, o1)
        sid = lax.axis_index("s")
        core = lax.axis_index("c")
        if split:
            w = sid
            npass = P // 2
        else:
            w = sid * _SC_CORES + core
            npass = P
        row0w = w * rows_w
        row0t = sid * VT

        pltpu.sync_copy(col_hbm.at[pl.ds(row0w * DEG, EW)], colv)
        pltpu.sync_copy(val_hbm.at[pl.ds(row0w * DEG, EW)], valv)

        def run_pass(j, _):
            p = j * 2 + core if split else j
            pc0 = p * CC

            pltpu.async_copy(
                x_hbm.at[pl.ds(row0t, VT), pl.ds(pc0, CC)],
                tab.at[pl.ds(row0t, VT)], ssem).wait()
            plsc.subcore_barrier()

            def x_desc(ci, slot):
                return pltpu.make_async_copy(
                    x0_hbm.at[pl.ds(row0w + ci * R, R), pl.ds(pc0, CC)],
                    x0v.at[slot], xsem[slot])

            def o_desc(ci, slot):
                return pltpu.make_async_copy(
                    outv.at[slot],
                    out_hbm.at[pl.ds(row0w + ci * R, R), pl.ds(pc0, CC)],
                    osem[slot])

            def g_descs(ci, slot):
                return [
                    pltpu.make_async_copy(
                        tab.at[colv.at[pl.ds(ci * E + s * ES, ES)]],
                        gath.at[slot].at[pl.ds(s * RS * DEG, ES)],
                        gsem[slot])
                    for s in range(NSUB)
                ]

            for d in g_descs(0, 0):
                d.start()
            if fuse:
                x_desc(0, 0).start()

            def step(ci, slot):
                @pl.when(ci + 1 < nch)
                def _():
                    for d in g_descs(ci + 1, 1 - slot):
                        d.start()
                    if fuse:
                        x_desc(ci + 1, 1 - slot).start()

                @pl.when(ci >= 2)
                def _():
                    o_desc(ci - 2, slot).wait()

                for d in g_descs(ci, slot):
                    d.wait()
                if fuse:
                    x_desc(ci, slot).wait()
                gb = gath.at[slot]
                ob = outv.at[slot]
                if fuse:
                    xb = x0v.at[slot]

                def rowpair(rr, _):
                    vv = valv[pl.ds(ci * E + rr * 16, 16)]
                    for half in range(2):
                        r = rr * 2 + half
                        base = r * DEG
                        vals = [vv[half * DEG + d] for d in range(DEG)]
                        if bf:
                            for cc in range(CC // 32):
                                cs = pl.ds(cc * 32, 32)
                                ga, go = plsc.unpack(
                                    gb[base, cs],
                                    format=plsc.PackFormat.INTERLEAVED)
                                aa = vals[0] * ga
                                ao = vals[0] * go
                                for d in range(1, DEG):
                                    ga, go = plsc.unpack(
                                        gb[base + d, cs],
                                        format=plsc.PackFormat.INTERLEAVED)
                                    aa = aa + vals[d] * ga
                                    ao = ao + vals[d] * go
                                if fuse:
                                    xa, xo = plsc.unpack(
                                        xb[r, cs],
                                        format=plsc.PackFormat.INTERLEAVED)
                                    aa = 2.0 * aa - xa
                                    ao = 2.0 * ao - xo
                                ob[r, cs] = plsc.pack(
                                    aa, ao, format=plsc.PackFormat.INTERLEAVED)
                        else:
                            for cc in range(NCC):
                                cs = pl.ds(cc * 16, 16)
                                acc = vals[0] * gb[base, cs]
                                for d in range(1, DEG):
                                    acc = acc + vals[d] * gb[base + d, cs]
                                if fuse:
                                    acc = 2.0 * acc - xb[r, cs]
                                ob[r, cs] = acc
                    return 0

                lax.fori_loop(0, R // 2, rowpair, 0)
                o_desc(ci, slot).start()

            def loop_j(j, _):
                step(j * 2, 0)
                step(j * 2 + 1, 1)
                return 0

            lax.fori_loop(0, nch // 2, loop_j, 0)
            o_desc(nch - 2, 0).wait()
            o_desc(nch - 1, 1).wait()
            plsc.subcore_barrier()
            return 0

        lax.fori_loop(0, npass, run_pass, 0)

    kparams = pltpu.CompilerParams(use_tc_tiling_on_sc=False,
                                   needs_layout_passes=not bf)
    if fuse:
        def run(X, col, val, Xprev):
            return pl.kernel(
                body,
                out_type=jax.ShapeDtypeStruct((V, C), adt),
                mesh=mesh,
                scratch_types=scratch,
                compiler_params=kparams,
            )(X, col, val, Xprev)
    else:
        def run(X, col, val):
            return pl.kernel(
                body,
                out_type=jax.ShapeDtypeStruct((V, C), adt),
                mesh=mesh,
                scratch_types=scratch,
                compiler_params=kparams,
            )(X, col, val)
    return run


def _linear_vm(x0, x1, x2, m0, m1, m2, b, pool, blk, odt=jnp.float32):
    V, Cin = x0.shape
    Cout = m0.shape[1]
    oblk = blk // 4 if pool else blk

    def body(x0_ref, x1_ref, x2_ref, m0_ref, m1_ref, m2_ref, b_ref, o_ref):
        xs = [x0_ref[...].astype(jnp.float32), x1_ref[...].astype(jnp.float32),
              x2_ref[...].astype(jnp.float32)]
        y = lax.dot_general(xs[0], m0_ref[...], (((1,), (0,)), ((), ())),
                            precision=_PREC, preferred_element_type=jnp.float32)
        y = y + lax.dot_general(xs[1], m1_ref[...], (((1,), (0,)), ((), ())),
                                precision=_PREC, preferred_element_type=jnp.float32)
        y = y + lax.dot_general(xs[2], m2_ref[...], (((1,), (0,)), ((), ())),
                                precision=_PREC, preferred_element_type=jnp.float32)
        y = y + b_ref[...]
        if pool:
            y = y.reshape(blk // 4, 4, Cout).max(axis=1)
        o_ref[...] = y.astype(odt)

    wspec = pl.BlockSpec((Cin, Cout), lambda i: (0, 0))
    return pl.pallas_call(
        body,
        grid=(V // blk,),
        in_specs=[
            pl.BlockSpec((blk, Cin), lambda i: (i, 0)),
            pl.BlockSpec((blk, Cin), lambda i: (i, 0)),
            pl.BlockSpec((blk, Cin), lambda i: (i, 0)),
            wspec, wspec, wspec,
            pl.BlockSpec((1, Cout), lambda i: (0, 0)),
        ],
        out_specs=pl.BlockSpec((oblk, Cout), lambda i: (i, 0)),
        out_shape=jax.ShapeDtypeStruct((V // 4 if pool else V, Cout), odt),
    )(x0, x1, x2, m0, m1, m2, b)


def _fc(Zt, Wfc1, bfc1, Wfc2, bfc2, ch=2048):
    K = Zt.shape[0]
    grid = K // ch

    def body(z_ref, w1_ref, b1_ref, w2_ref, b2_ref, o_ref, acc_ref):
        i = pl.program_id(0)

        @pl.when(i == 0)
        def _():
            acc_ref[...] = jnp.zeros_like(acc_ref)

        acc_ref[...] += lax.dot_general(
            w1_ref[...], z_ref[...], (((1,), (0,)), ((), ())),
            precision=_PREC, preferred_element_type=jnp.float32)

        @pl.when(i == grid - 1)
        def _():
            h1 = acc_ref[...] + b1_ref[...]
            out = lax.dot_general(w2_ref[...], h1, (((1,), (0,)), ((), ())),
                                  precision=_PREC,
                                  preferred_element_type=jnp.float32)
            o_ref[...] = out + b2_ref[...]

    return pl.pallas_call(
        body,
        grid=(grid,),
        in_specs=[
            pl.BlockSpec((ch, B), lambda i: (i, 0)),
            pl.BlockSpec((512, ch), lambda i: (0, i)),
            pl.BlockSpec((512, 1), lambda i: (0, 0)),
            pl.BlockSpec((NOUT, 512), lambda i: (0, 0)),
            pl.BlockSpec((NOUT, 1), lambda i: (0, 0)),
        ],
        out_specs=pl.BlockSpec((NOUT, B), lambda i: (0, 0)),
        out_shape=jax.ShapeDtypeStruct((NOUT, B), jnp.float32),
        scratch_shapes=[pltpu.VMEM((512, B), jnp.float32)],
    )(Zt, Wfc1, bfc1, Wfc2, bfc2)


def _wk(W):
    return [W[:, k::3].T for k in range(3)]


def kernel(x, W1, b1, W2, b2, W3, b3, W4, b4, Wfc1, bfc1, Wfc2, bfc2,
           L0_val, L1_val, L0_row, L0_col, L1_row, L1_col):
    del L0_row, L1_row

    spmm1 = _make_spmm(V0, B * FIN, 48, 64, False)
    spmm1f = _make_spmm(V0, B * FIN, 48, 64, True)
    bf16 = jnp.bfloat16
    spmm2 = _make_spmm(V0, B * 32, 64, 64, False, bf16)
    spmm2f = _make_spmm(V0, B * 32, 64, 64, True, bf16)
    spmm3 = _make_spmm(V1, B * 32, 256, 32, False, bf16)
    spmm3f = _make_spmm(V1, B * 32, 256, 32, True, bf16)
    spmm4 = _make_spmm(V1, B * 64, 256, 32, False, bf16)
    spmm4f = _make_spmm(V1, B * 64, 256, 32, True, bf16)

    eyeB = jnp.eye(B, dtype=jnp.float32)

    def lift_bm(wk):
        return jnp.kron(eyeB, wk)

    def lift_om(wk):
        F, O = wk.shape
        return jnp.einsum('fo,bc->bfoc', wk, eyeB).reshape(B * F, O * B)

    X0 = jnp.transpose(x, (1, 0, 2)).reshape(V0, B * FIN)
    X1 = spmm1(X0, L0_col, L0_val)
    X2 = spmm1f(X1, L0_col, L0_val, X0)
    k0, k1, k2 = _wk(W1)
    H = _linear_vm(X0, X1, X2, lift_bm(k0), lift_bm(k1), lift_bm(k2),
                   jnp.tile(b1, B).reshape(1, -1), False, 2048, jnp.bfloat16)

    X1 = spmm2(H, L0_col, L0_val)
    X2 = spmm2f(X1, L0_col, L0_val, H)
    k0, k1, k2 = _wk(W2)
    H = _linear_vm(H, X1, X2, lift_bm(k0), lift_bm(k1), lift_bm(k2),
                   jnp.tile(b2, B).reshape(1, -1), True, 1024, jnp.bfloat16)

    X1 = spmm3(H, L1_col, L1_val)
    X2 = spmm3f(X1, L1_col, L1_val, H)
    k0, k1, k2 = _wk(W3)
    H = _linear_vm(H, X1, X2, lift_bm(k0), lift_bm(k1), lift_bm(k2),
                   jnp.tile(b3, B).reshape(1, -1), False, 1024, jnp.bfloat16)

    X1 = spmm4(H, L1_col, L1_val)
    X2 = spmm4f(X1, L1_col, L1_val, H)
    k0, k1, k2 = _wk(W4)
    Z = _linear_vm(H, X1, X2, lift_om(k0), lift_om(k1), lift_om(k2),
                   jnp.repeat(b4, B).reshape(1, -1), True, 512)

    Zt = Z.reshape(B * 1024 * 4, B)
    outT = _fc(Zt, Wfc1, bfc1.reshape(-1, 1), Wfc2, bfc2.reshape(-1, 1))
    return outT.T
